# baseline (speedup 1.0000x reference)
import dataclasses
import functools

import jax
import jax.numpy as jnp
from jax import lax
from jax.experimental import pallas as pl
from jax.experimental.pallas import tpu as pltpu
from jax.experimental.pallas import tpu_sc as plsc

F32 = jnp.float32
I32 = jnp.int32
_NC, _NS, _LN = 2, 16, 16
_NW = _NC * _NS
_CHUNK = 128
_PREC = lax.Precision.DEFAULT


def _silu(x):
    return x * jax.nn.sigmoid(x)


def _dot(a, b):
    return lax.dot_general(a, b, (((1,), (0,)), ((), ())),
                           precision=_PREC, preferred_element_type=F32)


def _ceil_to(n, m):
    return -(-n // m) * m


def _rpad(x, rows):
    return jnp.pad(x, ((0, rows - x.shape[0]),) + ((0, 0),) * (x.ndim - 1))


def _cpad(x, cols):
    return jnp.pad(x, ((0, 0), (0, cols - x.shape[1])))


def _ipad(x, n, val):
    return jnp.concatenate([x.astype(I32), jnp.full((n - x.shape[0],), val, I32)])


def _blk_rows(r):
    return 2048 if r % 2048 == 0 else (1024 if r % 1024 == 0 else r)


def _sc_params():
    cp = pltpu.CompilerParams()
    if "use_tc_tiling_on_sc" in pltpu.CompilerParams.__dataclass_fields__:
        cp = dataclasses.replace(cp, use_tc_tiling_on_sc=False)
    return cp


def _sc_mesh():
    return plsc.VectorSubcoreMesh(core_axis_name="c", subcore_axis_name="s",
                                  num_cores=_NC, num_subcores=_NS)


@functools.lru_cache(None)
def _sc_gather2(e_pad, d, nsub):
    epw = e_pad // _NW
    nch = epw // _CHUNK
    ngrp = d // _LN

    def body(ta, tb, ia, ib, z, ia_v, ib_v, ra_v, rb_v, sa, sb):
        wid = lax.axis_index("s") * _NC + lax.axis_index("c")
        base0 = wid * epw

        @pl.loop(0, nch)
        def _chunk(g):
            base = base0 + g * _CHUNK
            pltpu.sync_copy(ia.at[pl.ds(base, _CHUNK)], ia_v)
            pltpu.sync_copy(ib.at[pl.ds(base, _CHUNK)], ib_v)
            cpa = pltpu.async_copy(ta.at[ia_v], ra_v, sa)
            cpb = pltpu.async_copy(tb.at[ib_v], rb_v, sb)
            cpa.wait()
            cpb.wait()

            @pl.loop(0, _CHUNK)
            def _row(r):
                for j in range(ngrp):
                    sl = pl.ds(j * _LN, _LN)
                    if j < ngrp - nsub:
                        ra_v[r, sl] = ra_v[r, sl] + rb_v[r, sl]
                    else:
                        ra_v[r, sl] = ra_v[r, sl] - rb_v[r, sl]

            pltpu.sync_copy(ra_v, z.at[pl.ds(base, _CHUNK)])

    return pl.kernel(
        body,
        out_type=jax.ShapeDtypeStruct((e_pad, d), F32),
        mesh=_sc_mesh(),
        compiler_params=_sc_params(),
        scratch_types=[
            pltpu.VMEM((_CHUNK,), I32),
            pltpu.VMEM((_CHUNK,), I32),
            pltpu.VMEM((_CHUNK, d), F32),
            pltpu.VMEM((_CHUNK, d), F32),
            pltpu.SemaphoreType.DMA,
            pltpu.SemaphoreType.DMA,
        ],
    )


@functools.lru_cache(None)
def _sc_gather1(e_pad, d):
    epw = e_pad // _NW
    nch = epw // _CHUNK

    def body(ta, ia, z, ia_v, ra_v, sa):
        wid = lax.axis_index("s") * _NC + lax.axis_index("c")
        base0 = wid * epw

        @pl.loop(0, nch)
        def _chunk(g):
            base = base0 + g * _CHUNK
            pltpu.sync_copy(ia.at[pl.ds(base, _CHUNK)], ia_v)
            pltpu.async_copy(ta.at[ia_v], ra_v, sa).wait()
            pltpu.sync_copy(ra_v, z.at[pl.ds(base, _CHUNK)])

    return pl.kernel(
        body,
        out_type=jax.ShapeDtypeStruct((e_pad, d), F32),
        mesh=_sc_mesh(),
        compiler_params=_sc_params(),
        scratch_types=[
            pltpu.VMEM((_CHUNK,), I32),
            pltpu.VMEM((_CHUNK, d), F32),
            pltpu.SemaphoreType.DMA,
        ],
    )


@functools.lru_cache(None)
def _sc_scatter_add(e_pad, d, n_pad):
    epw = e_pad // _NW
    nch = epw // _CHUNK
    rpt = n_pad // _NS

    def body(m, idx, zrs, out, idx_v, m_v, acc):
        c = lax.axis_index("c")
        s = lax.axis_index("s")
        wid = s * _NC + c
        row0 = s * rpt
        pltpu.sync_copy(zrs.at[pl.ds(row0, rpt)], acc.at[pl.ds(row0, rpt)])
        plsc.subcore_barrier()

        base0 = wid * epw

        @pl.loop(0, nch)
        def _chunk(g):
            base = base0 + g * _CHUNK
            pltpu.sync_copy(idx.at[pl.ds(base, _CHUNK)], idx_v)
            pltpu.sync_copy(m.at[pl.ds(base, _CHUNK)], m_v)
            pltpu.sync_copy(m_v, acc.at[idx_v], add=True)

        plsc.subcore_barrier()
        pltpu.sync_copy(acc.at[pl.ds(row0, rpt)],
                        out.at[pl.ds(c * n_pad + row0, rpt)])

    return pl.kernel(
        body,
        out_type=jax.ShapeDtypeStruct((2 * n_pad, d), F32),
        mesh=_sc_mesh(),
        compiler_params=_sc_params(),
        scratch_types=[
            pltpu.VMEM((_CHUNK,), I32),
            pltpu.VMEM((_CHUNK, d), F32),
            pltpu.VMEM_SHARED((n_pad, d), F32),
        ],
    )


@functools.lru_cache(None)
def _sc_fused_lig(e_pad, n_pad):
    C = 64
    epw = e_pad // _NW
    nch = epw // C
    rpt = n_pad // _NS

    def body(ta, tb, ee, ia, ib, zrs, out,
             ia0, ia1, ib0, ib1, ra0, ra1, rb0, rb1, ee_v, acc,
             gs0, gs1, ss0, ss1):
        c = lax.axis_index("c")
        s = lax.axis_index("s")
        wid = s * _NC + c
        row0 = s * rpt
        pltpu.sync_copy(zrs.at[pl.ds(row0, rpt)], acc.at[pl.ds(row0, rpt)])
        plsc.subcore_barrier()
        base0 = wid * epw

        ia_v = [ia0, ia1]
        ib_v = [ib0, ib1]
        ra_v = [ra0, ra1]
        rb_v = [rb0, rb1]
        gsem = [gs0, gs1]
        ssem = [ss0, ss1]

        def issue(b, g):
            base = base0 + g * C
            pltpu.sync_copy(ia.at[pl.ds(base, C)], ia_v[b])
            pltpu.sync_copy(ib.at[pl.ds(base, C)], ib_v[b])
            pltpu.async_copy(ta.at[ia_v[b]], ra_v[b], gsem[b])
            pltpu.async_copy(tb.at[ib_v[b]], rb_v[b], gsem[b])

        def wait_gather(b):
            for dst in (ra_v[b], rb_v[b]):
                pltpu.make_async_copy(zrs.at[pl.ds(0, C)], dst, gsem[b]).wait()

        def compute(b):
            @pl.loop(0, C)
            def _row(r):
                for j in range(8):
                    sl = pl.ds(j * _LN, _LN)
                    z = ra_v[b][r, sl] + rb_v[b][r, sl] + ee_v[r, sl]
                    ra_v[b][r, sl] = z / (1.0 + jnp.exp(-z))

        def scatter(b):
            pltpu.async_copy(ra_v[b], acc.at[ia_v[b]], ssem[b], add=True)

        def wait_scatter(b):
            pltpu.make_async_copy(ra_v[b], acc.at[ia_v[b]], ssem[b]).wait()

        issue(0, 0)

        @pl.loop(0, nch, step=2)
        def _pair(g):
            for b in (0, 1):
                ge = g + b
                nb2 = 1 - b
                wait_gather(b)
                pltpu.sync_copy(ee.at[pl.ds(base0 + ge * C, C)], ee_v)

                @pl.when(jnp.logical_and(ge >= 1, ge + 1 < nch))
                def _w():
                    wait_scatter(nb2)

                @pl.when(ge + 1 < nch)
                def _i():
                    issue(nb2, ge + 1)

                compute(b)
                scatter(b)

        wait_scatter(0)
        wait_scatter(1)
        plsc.subcore_barrier()
        pltpu.sync_copy(acc.at[pl.ds(row0, rpt)],
                        out.at[pl.ds(c * n_pad + row0, rpt)])

    return pl.kernel(
        body,
        out_type=jax.ShapeDtypeStruct((2 * n_pad, 128), F32),
        mesh=_sc_mesh(),
        compiler_params=_sc_params(),
        scratch_types=(
            [pltpu.VMEM((64,), I32)] * 4
            + [pltpu.VMEM((64, 128), F32)] * 5
            + [pltpu.VMEM_SHARED((n_pad, 128), F32)]
            + [pltpu.SemaphoreType.DMA] * 4
        ),
    )


@functools.lru_cache(None)
def _sc_fused_aa(e_pad, n_pad):
    C = 64
    epw = e_pad // _NW
    nch = epw // C
    rpt = n_pad // _NS

    def body(ta, ee, ia, idx, zrs, out,
             ia0, ia1, id0, id1, ra0, ra1, ee_v, m0, m1, acc,
             gs0, gs1, ss0, ss1):
        c = lax.axis_index("c")
        s = lax.axis_index("s")
        wid = s * _NC + c
        row0 = s * rpt
        pltpu.sync_copy(zrs.at[pl.ds(row0, rpt)], acc.at[pl.ds(row0, rpt)])
        base0 = wid * epw

        ia_v = [ia0, ia1]
        id_v = [id0, id1]
        ra_v = [ra0, ra1]
        m_v = [m0, m1]
        gsem = [gs0, gs1]
        ssem = [ss0, ss1]

        lane = lax.iota(I32, _LN)
        cnt1 = jnp.where(lane == 0, 1.0, 0.0).astype(F32)
        for b in (0, 1):
            @pl.loop(0, C)
            def _row(r):
                m_v[b][r, pl.ds(128, _LN)] = cnt1
        plsc.subcore_barrier()

        def issue(b, g):
            base = base0 + g * C
            pltpu.sync_copy(ia.at[pl.ds(base, C)], ia_v[b])
            pltpu.sync_copy(idx.at[pl.ds(base, C)], id_v[b])
            pltpu.async_copy(ta.at[ia_v[b]], ra_v[b], gsem[b])

        def wait_gather(b):
            pltpu.make_async_copy(zrs.at[pl.ds(0, C), pl.ds(0, 128)],
                                  ra_v[b], gsem[b]).wait()

        def compute(b):
            @pl.loop(0, C)
            def _row(r):
                for j in range(8):
                    sl = pl.ds(j * _LN, _LN)
                    z = ra_v[b][r, sl] + ee_v[r, sl]
                    m_v[b][r, sl] = z / (1.0 + jnp.exp(-z))

        def scatter(b):
            pltpu.async_copy(m_v[b], acc.at[id_v[b]], ssem[b], add=True)

        def wait_scatter(b):
            pltpu.make_async_copy(m_v[b], acc.at[id_v[b]], ssem[b]).wait()

        issue(0, 0)

        @pl.loop(0, nch, step=2)
        def _pair(g):
            for b in (0, 1):
                ge = g + b
                nb2 = 1 - b
                wait_gather(b)
                pltpu.sync_copy(ee.at[pl.ds(base0 + ge * C, C)], ee_v)

                @pl.when(jnp.logical_and(ge >= 1, ge + 1 < nch))
                def _w():
                    wait_scatter(nb2)

                @pl.when(ge + 1 < nch)
                def _i():
                    issue(nb2, ge + 1)

                compute(b)
                scatter(b)

        wait_scatter(0)
        wait_scatter(1)
        plsc.subcore_barrier()
        pltpu.sync_copy(acc.at[pl.ds(row0, rpt)],
                        out.at[pl.ds(c * n_pad + row0, rpt)])

    return pl.kernel(
        body,
        out_type=jax.ShapeDtypeStruct((2 * n_pad, 144), F32),
        mesh=_sc_mesh(),
        compiler_params=_sc_params(),
        scratch_types=(
            [pltpu.VMEM((64,), I32)] * 4
            + [pltpu.VMEM((64, 128), F32)] * 3
            + [pltpu.VMEM((64, 144), F32)] * 2
            + [pltpu.VMEM_SHARED((n_pad, 144), F32)]
            + [pltpu.SemaphoreType.DMA] * 4
        ),
    )


@functools.lru_cache(None)
def _tc_linear_call(r, k, f, act, blk):
    def body(x_ref, w_ref, b_ref, o_ref):
        y = _dot(x_ref[...], w_ref[...]) + b_ref[...]
        o_ref[...] = _silu(y) if act else y

    return pl.pallas_call(
        body,
        grid=(r // blk,),
        in_specs=[
            pl.BlockSpec((blk, k), lambda i: (i, 0)),
            pl.BlockSpec((k, f), lambda i: (0, 0)),
            pl.BlockSpec((1, f), lambda i: (0, 0)),
        ],
        out_specs=pl.BlockSpec((blk, f), lambda i: (i, 0)),
        out_shape=jax.ShapeDtypeStruct((r, f), F32),
    )


def _tc_linear(x, w, b, act):
    r, k = x.shape
    f = w.shape[1]
    return _tc_linear_call(r, k, f, act, _blk_rows(r))(x, w, b.reshape(1, f))


@functools.lru_cache(None)
def _tc_msg_lig_call(e, blk):
    def body(z_ref, ea_ref, w_ref, o_ref):
        o_ref[...] = _silu(z_ref[...] + _dot(ea_ref[...], w_ref[...]))

    return pl.pallas_call(
        body,
        grid=(e // blk,),
        in_specs=[
            pl.BlockSpec((blk, 128), lambda i: (i, 0)),
            pl.BlockSpec((blk, 16), lambda i: (i, 0)),
            pl.BlockSpec((16, 128), lambda i: (0, 0)),
        ],
        out_specs=pl.BlockSpec((blk, 128), lambda i: (i, 0)),
        out_shape=jax.ShapeDtypeStruct((e, 128), F32),
    )


@functools.lru_cache(None)
def _tc_msg_cx_call(e, blk):
    def body(z_ref, ea_ref, we_ref, wd2_ref, wx_ref, bx_ref, o_ref):
        zz = z_ref[...]
        z128 = zz[:, :128]
        rel = zz[:, 128:131]
        d2 = jnp.sum(rel * rel, axis=1, keepdims=True)
        d2b = d2.astype(jnp.bfloat16).astype(F32)
        wd2 = wd2_ref[...].astype(jnp.bfloat16).astype(F32)
        mm = _silu(z128 + d2b * wd2 + _dot(ea_ref[...], we_ref[...]))
        coef = jnp.tanh(_dot(mm, wx_ref[...])[:, 0:1] + bx_ref[...][:, 0:1])
        relc = rel * coef
        ones = jnp.ones((blk, 1), F32)
        zer = jnp.zeros((blk, 12), F32)
        o_ref[...] = jnp.concatenate([mm, relc, ones, zer], axis=1)

    return pl.pallas_call(
        body,
        grid=(e // blk,),
        in_specs=[
            pl.BlockSpec((blk, 144), lambda i: (i, 0)),
            pl.BlockSpec((blk, 8), lambda i: (i, 0)),
            pl.BlockSpec((8, 128), lambda i: (0, 0)),
            pl.BlockSpec((1, 128), lambda i: (0, 0)),
            pl.BlockSpec((128, 128), lambda i: (0, 0)),
            pl.BlockSpec((1, 128), lambda i: (0, 0)),
        ],
        out_specs=pl.BlockSpec((blk, 144), lambda i: (i, 0)),
        out_shape=jax.ShapeDtypeStruct((e, 144), F32),
    )


@functools.lru_cache(None)
def _tc_msg_aa_call(e, blk):
    def body(z_ref, es_ref, ev_ref, we_ref, wv_ref, o_ref):
        ev = ev_ref[...][:, 0:3]
        evn = jnp.sqrt(jnp.sum(ev * ev, axis=1, keepdims=True) + 1e-8)
        evnb = evn.astype(jnp.bfloat16).astype(F32)
        wvb = wv_ref[...].astype(jnp.bfloat16).astype(F32)
        ma = _silu(z_ref[...] + _dot(es_ref[...], we_ref[...])
                   + evnb * wvb)
        ones = jnp.ones((blk, 1), F32)
        zer = jnp.zeros((blk, 15), F32)
        o_ref[...] = jnp.concatenate([ma, ones, zer], axis=1)

    return pl.pallas_call(
        body,
        grid=(e // blk,),
        in_specs=[
            pl.BlockSpec((blk, 128), lambda i: (i, 0)),
            pl.BlockSpec((blk, 32), lambda i: (i, 0)),
            pl.BlockSpec((blk, 8), lambda i: (i, 0)),
            pl.BlockSpec((32, 128), lambda i: (0, 0)),
            pl.BlockSpec((1, 128), lambda i: (0, 0)),
        ],
        out_specs=pl.BlockSpec((blk, 144), lambda i: (i, 0)),
        out_shape=jax.ShapeDtypeStruct((e, 144), F32),
    )


@functools.lru_cache(None)
def _tc_upd_call(n, blk):
    nb = n // blk

    def body(h_ref, s0_ref, s1_ref, wh_ref, wa_ref, b_ref, o_ref):
        agg = s0_ref[...] + s1_ref[...]
        o_ref[...] = _silu(_dot(h_ref[...], wh_ref[...])
                           + _dot(agg, wa_ref[...]) + b_ref[...])

    return pl.pallas_call(
        body,
        grid=(nb,),
        in_specs=[
            pl.BlockSpec((blk, 128), lambda i: (i, 0)),
            pl.BlockSpec((blk, 128), lambda i: (i, 0)),
            pl.BlockSpec((blk, 128), lambda i: (i + nb, 0)),
            pl.BlockSpec((128, 128), lambda i: (0, 0)),
            pl.BlockSpec((128, 128), lambda i: (0, 0)),
            pl.BlockSpec((1, 128), lambda i: (0, 0)),
        ],
        out_specs=pl.BlockSpec((blk, 128), lambda i: (i, 0)),
        out_shape=jax.ShapeDtypeStruct((n, 128), F32),
    )


@functools.lru_cache(None)
def _tc_upd_cx_call(n, blk):
    nb = n // blk

    def body(h_ref, p_ref, s0_ref, s1_ref, wh_ref, wa_ref, b_ref,
             oh_ref, op_ref):
        agg = s0_ref[...] + s1_ref[...]
        aggm = agg[:, :128]
        extras = agg[:, 128:144]
        cnt = extras[:, 3:4]
        inv = 1.0 / jnp.maximum(cnt, 1.0)
        lane = lax.broadcasted_iota(I32, (blk, 16), 1)
        op_ref[...] = p_ref[...] + jnp.where(lane < 3, extras * inv, 0.0)
        oh_ref[...] = _silu(_dot(h_ref[...], wh_ref[...])
                            + _dot(aggm, wa_ref[...]) + b_ref[...])

    return pl.pallas_call(
        body,
        grid=(nb,),
        in_specs=[
            pl.BlockSpec((blk, 128), lambda i: (i, 0)),
            pl.BlockSpec((blk, 16), lambda i: (i, 0)),
            pl.BlockSpec((blk, 144), lambda i: (i, 0)),
            pl.BlockSpec((blk, 144), lambda i: (i + nb, 0)),
            pl.BlockSpec((128, 128), lambda i: (0, 0)),
            pl.BlockSpec((128, 128), lambda i: (0, 0)),
            pl.BlockSpec((1, 128), lambda i: (0, 0)),
        ],
        out_specs=[
            pl.BlockSpec((blk, 128), lambda i: (i, 0)),
            pl.BlockSpec((blk, 16), lambda i: (i, 0)),
        ],
        out_shape=[
            jax.ShapeDtypeStruct((n, 128), F32),
            jax.ShapeDtypeStruct((n, 16), F32),
        ],
    )


@functools.lru_cache(None)
def _tc_upd_aa_call(n, blk):
    nb = n // blk

    def body(h_ref, s0_ref, s1_ref, wh_ref, wa_ref, b_ref, o_ref):
        agg = s0_ref[...] + s1_ref[...]
        cnt = agg[:, 128:129]
        aggmean = agg[:, :128] / jnp.maximum(cnt, 1.0)
        o_ref[...] = _silu(_dot(h_ref[...], wh_ref[...])
                           + _dot(aggmean, wa_ref[...]) + b_ref[...])

    return pl.pallas_call(
        body,
        grid=(nb,),
        in_specs=[
            pl.BlockSpec((blk, 128), lambda i: (i, 0)),
            pl.BlockSpec((blk, 144), lambda i: (i, 0)),
            pl.BlockSpec((blk, 144), lambda i: (i + nb, 0)),
            pl.BlockSpec((128, 128), lambda i: (0, 0)),
            pl.BlockSpec((128, 128), lambda i: (0, 0)),
            pl.BlockSpec((1, 128), lambda i: (0, 0)),
        ],
        out_specs=pl.BlockSpec((blk, 128), lambda i: (i, 0)),
        out_shape=jax.ShapeDtypeStruct((n, 128), F32),
    )


@functools.lru_cache(None)
def _tc_embed_aa_call(n, blk):
    def body(s8_ref, sq_ref, v16_ref, ws_ref, woh_ref, wv_ref, b_ref, o_ref):
        sq = sq_ref[...][:, 0:1]
        i20 = lax.broadcasted_iota(I32, (1, 20), 1).astype(F32)
        oh = (sq == i20).astype(F32)
        v = v16_ref[...]

        def nrm(k):
            sl = v[:, 3 * k:3 * k + 3]
            return jnp.sqrt(jnp.sum(sl * sl, axis=1, keepdims=True) + 1e-8)

        vn = jnp.concatenate([nrm(0), nrm(1), nrm(2)], axis=1)
        o_ref[...] = _silu(_dot(s8_ref[...], ws_ref[...])
                           + _dot(oh, woh_ref[...])
                           + _dot(vn, wv_ref[...]) + b_ref[...])

    return pl.pallas_call(
        body,
        grid=(n // blk,),
        in_specs=[
            pl.BlockSpec((blk, 8), lambda i: (i, 0)),
            pl.BlockSpec((blk, 8), lambda i: (i, 0)),
            pl.BlockSpec((blk, 16), lambda i: (i, 0)),
            pl.BlockSpec((8, 128), lambda i: (0, 0)),
            pl.BlockSpec((20, 128), lambda i: (0, 0)),
            pl.BlockSpec((3, 128), lambda i: (0, 0)),
            pl.BlockSpec((1, 128), lambda i: (0, 0)),
        ],
        out_specs=pl.BlockSpec((blk, 128), lambda i: (i, 0)),
        out_shape=jax.ShapeDtypeStruct((n, 128), F32),
    )


@functools.lru_cache(None)
def _tc_eea_call(e, blk):
    def body(es_ref, ev_ref, we_ref, wv_ref, o_ref):
        ev = ev_ref[...][:, 0:3]
        evn = jnp.sqrt(jnp.sum(ev * ev, axis=1, keepdims=True) + 1e-8)
        evnb = evn.astype(jnp.bfloat16).astype(F32)
        wvb = wv_ref[...].astype(jnp.bfloat16).astype(F32)
        o_ref[...] = _dot(es_ref[...], we_ref[...]) + evnb * wvb

    return pl.pallas_call(
        body,
        grid=(e // blk,),
        in_specs=[
            pl.BlockSpec((blk, 32), lambda i: (i, 0)),
            pl.BlockSpec((blk, 8), lambda i: (i, 0)),
            pl.BlockSpec((32, 128), lambda i: (0, 0)),
            pl.BlockSpec((1, 128), lambda i: (0, 0)),
        ],
        out_specs=pl.BlockSpec((blk, 128), lambda i: (i, 0)),
        out_shape=jax.ShapeDtypeStruct((e, 128), F32),
    )


@functools.lru_cache(None)
def _tc_pool_call(n, blk):
    def body(x_ref, bid_ref, s_ref, c_ref):
        @pl.when(pl.program_id(0) == 0)
        def _init():
            s_ref[...] = jnp.zeros_like(s_ref)
            c_ref[...] = jnp.zeros_like(c_ref)

        bid = bid_ref[...]
        i16 = lax.broadcasted_iota(I32, (16, blk), 0)
        one_t = (i16 == bid).astype(F32)
        s_ref[...] += lax.dot_general(one_t, x_ref[...],
                                      (((1,), (0,)), ((), ())),
                                      precision=lax.Precision.HIGHEST,
                                      preferred_element_type=F32)
        csum = jnp.sum(one_t, axis=1, keepdims=True)
        c_ref[...] += jnp.broadcast_to(csum, (16, 128))

    return pl.pallas_call(
        body,
        grid=(n // blk,),
        in_specs=[
            pl.BlockSpec((blk, 128), lambda i: (i, 0)),
            pl.BlockSpec((1, blk), lambda i: (0, i)),
        ],
        out_specs=[
            pl.BlockSpec((16, 128), lambda i: (0, 0)),
            pl.BlockSpec((16, 128), lambda i: (0, 0)),
        ],
        out_shape=[
            jax.ShapeDtypeStruct((16, 128), F32),
            jax.ShapeDtypeStruct((16, 128), F32),
        ],
    )


@functools.lru_cache(None)
def _tc_head_call():
    def body(sl_ref, cl_ref, sa_ref, ca_ref, sc_ref, cc_ref,
             wi_ref, bi_ref, wg1_ref, bg1_ref, wg2_ref, bg2_ref,
             o1_ref, o2_ref):
        p_l = sl_ref[...] / jnp.maximum(cl_ref[...], 1.0)
        p_a = sa_ref[...] / jnp.maximum(ca_ref[...], 1.0)
        x_c = sc_ref[...] / jnp.maximum(cc_ref[...], 1.0)
        inter = _silu(_dot(jnp.concatenate([p_l, p_a], axis=1), wi_ref[...])
                      + bi_ref[...])

        def g(v):
            h1 = jnp.maximum(_dot(v, wg1_ref[...]) + bg1_ref[...], 0.0)
            return _dot(h1, wg2_ref[...]) + bg2_ref[...]

        o1_ref[...] = g(inter)
        o2_ref[...] = g(x_c)

    return pl.pallas_call(
        body,
        out_shape=[
            jax.ShapeDtypeStruct((16, 128), F32),
            jax.ShapeDtypeStruct((16, 128), F32),
        ],
    )


def kernel(x_l, edge_attr_l, x_c, pos_c, edge_attr_c, node_s, node_v,
           edge_s, edge_v, params, edge_index_l, batch_l, edge_index_c,
           batch_c, seq, edge_index_aa, batch_aa):
    p = params
    n_l, n_c, n_a = x_l.shape[0], x_c.shape[0], node_s.shape[0]
    e_l, e_c, e_a = (edge_index_l.shape[1], edge_index_c.shape[1],
                     edge_index_aa.shape[1])
    np_l, np_c, np_a = (_ceil_to(n_l, 2048), _ceil_to(n_c, 2048),
                        _ceil_to(n_a, 2048))
    ep_l, ep_c, ep_a = (_ceil_to(e_l, 4096), _ceil_to(e_c, 4096),
                        _ceil_to(e_a, 4096))

    srcl_g = _ipad(edge_index_l[0], ep_l, 0)
    dstl_g = _ipad(edge_index_l[1], ep_l, 0)
    dstl_s = _ipad(edge_index_l[1], ep_l, n_l)
    srcc_g = _ipad(edge_index_c[0], ep_c, 0)
    dstc_g = _ipad(edge_index_c[1], ep_c, 0)
    dstc_s = _ipad(edge_index_c[1], ep_c, n_c)
    saa_g = _ipad(edge_index_aa[0], ep_a, 0)
    daa_s = _ipad(edge_index_aa[1], ep_a, n_a)

    ea_l = _rpad(edge_attr_l, ep_l)
    ea_c = _cpad(_rpad(edge_attr_c, ep_c), 8)
    es_a = _rpad(edge_s, ep_a)
    ev_a = _cpad(_rpad(edge_v.reshape(e_a, 3), ep_a), 8)

    bid_l = _ipad(batch_l, np_l, 63).reshape(1, np_l)
    bid_c = _ipad(batch_c, np_c, 63).reshape(1, np_c)
    bid_a = _ipad(batch_aa, np_a, 63).reshape(1, np_a)

    zeros_l = jnp.zeros((np_l, 128), F32)
    zeros_c = jnp.zeros((np_c, 144), F32)
    zeros_a = jnp.zeros((np_a, 144), F32)

    def dep(idx_arr, t):
        if t is None:
            return idx_arr
        out, _ = lax.optimization_barrier((idx_arr, t))
        return out

    tok = None

    h = _tc_linear(_rpad(x_l, np_l), p['W_lig'], p['b_lig'], True)
    hc = _tc_linear(_rpad(x_c, np_c), p['W_cx'], p['b_cx'], True)
    pos16 = _cpad(_rpad(pos_c, np_c), 16)
    ns8 = _cpad(_rpad(node_s, np_a), 8)
    sq8 = _cpad(_rpad(seq.astype(F32).reshape(n_a, 1), np_a), 8)
    v16 = _cpad(_rpad(node_v.reshape(n_a, 9), np_a), 16)
    w_aa = p['W_aa']
    ws8 = jnp.pad(w_aa[:6], ((0, 2), (0, 0)))
    ha = _tc_embed_aa_call(np_a, 2048)(ns8, sq8, v16, ws8, w_aa[6:26],
                                       w_aa[26:29],
                                       p['b_aa'].reshape(1, 128))

    for i in ('1', '2', '3'):
        wm = p['Wm' + i]
        w2 = jnp.concatenate([wm[:128], wm[128:256]], axis=1)
        b2 = jnp.concatenate([p['bm' + i], jnp.zeros((128,), F32)])
        ab = _tc_linear(h, w2, b2, False)
        ee = _tc_linear(ea_l, wm[256:], jnp.zeros((128,), F32), False)
        s = _sc_fused_lig(ep_l, np_l)(ab[:, :128], ab[:, 128:], ee,
                                      dep(dstl_s, tok), srcl_g, zeros_l)
        tok = s[0, 0]
        wu = p['Wu' + i]
        h = _tc_upd_call(np_l, 2048)(h, s, s, wu[:128], wu[128:],
                                     p['bu' + i].reshape(1, 128))
    s_l, c_l = _tc_pool_call(np_l, 2048)(h, bid_l)

    for i in ('1', '2', '3'):
        we = p['We' + i]
        w2 = jnp.concatenate([we[:128], we[128:256]], axis=1)
        b2 = jnp.concatenate([p['be' + i], jnp.zeros((128,), F32)])
        ab = _tc_linear(hc, w2, b2, False)
        ta = jnp.concatenate([ab[:, :128], pos16], axis=1)
        tb = jnp.concatenate([ab[:, 128:], pos16], axis=1)
        z = _sc_gather2(ep_c, 144, 1)(ta, tb, dstc_g, srcc_g)
        we_e = jnp.pad(we[257:261], ((0, 4), (0, 0)))
        wx = jnp.pad(p['Wx' + i], ((0, 0), (0, 127)))
        bx = jnp.broadcast_to(p['bx' + i].reshape(1, 1), (1, 128))
        m = _tc_msg_cx_call(ep_c, 2048)(z, ea_c, we_e, we[256:257], wx, bx)
        s = _sc_scatter_add(ep_c, 144, np_c)(m, dep(dstc_s, tok), zeros_c)
        tok = s[0, 0]
        wh = p['Wh' + i]
        hc, pos16 = _tc_upd_cx_call(np_c, 2048)(
            hc, pos16, s, s, wh[:128], wh[128:],
            p['bh' + i].reshape(1, 128))
    s_c, c_c = _tc_pool_call(np_c, 2048)(hc, bid_c)

    aa_t = _tc_linear(ha, p['W_amsg'][:128], p['b_amsg'], False)
    ee_a = _tc_eea_call(ep_a, 2048)(es_a, ev_a, p['W_amsg'][128:160],
                                    p['W_amsg'][160:161])
    s = _sc_fused_aa(ep_a, np_a)(aa_t, ee_a, saa_g, dep(daa_s, tok), zeros_a)
    wv = p['W_aupd']
    ha = _tc_upd_aa_call(np_a, 2048)(ha, s, s, wv[:128], wv[128:],
                                     p['b_aupd'].reshape(1, 128))
    s_a, c_a = _tc_pool_call(np_a, 2048)(ha, bid_a)

    wg2 = jnp.pad(p['W_g2'], ((0, 0), (0, 127)))
    bg2 = jnp.broadcast_to(p['b_g2'].reshape(1, 1), (1, 128))
    o1, o2 = _tc_head_call()(s_l, c_l, s_a, c_a, s_c, c_c,
                             p['W_int'], p['b_int'].reshape(1, 128),
                             p['W_g1'], p['b_g1'].reshape(1, 256),
                             wg2, bg2)
    return o1[:, :1], o2[:, :1]


# speedup vs baseline: 1.0759x; 1.0759x over previous
import dataclasses
import functools

import jax
import jax.numpy as jnp
from jax import lax
from jax.experimental import pallas as pl
from jax.experimental.pallas import tpu as pltpu
from jax.experimental.pallas import tpu_sc as plsc

F32 = jnp.float32
I32 = jnp.int32
_NC, _NS, _LN = 2, 16, 16
_NW = _NC * _NS
_CHUNK = 128
_PREC = lax.Precision.DEFAULT


def _silu(x):
    return x * jax.nn.sigmoid(x)


def _dot(a, b):
    return lax.dot_general(a, b, (((1,), (0,)), ((), ())),
                           precision=_PREC, preferred_element_type=F32)


def _ceil_to(n, m):
    return -(-n // m) * m


def _rpad(x, rows):
    return jnp.pad(x, ((0, rows - x.shape[0]),) + ((0, 0),) * (x.ndim - 1))


def _cpad(x, cols):
    return jnp.pad(x, ((0, 0), (0, cols - x.shape[1])))


def _ipad(x, n, val):
    return jnp.concatenate([x.astype(I32), jnp.full((n - x.shape[0],), val, I32)])


def _blk_rows(r):
    return 2048 if r % 2048 == 0 else (1024 if r % 1024 == 0 else r)


def _sc_params():
    cp = pltpu.CompilerParams()
    if "use_tc_tiling_on_sc" in pltpu.CompilerParams.__dataclass_fields__:
        cp = dataclasses.replace(cp, use_tc_tiling_on_sc=False)
    return cp


def _sc_mesh():
    return plsc.VectorSubcoreMesh(core_axis_name="c", subcore_axis_name="s",
                                  num_cores=_NC, num_subcores=_NS)


@functools.lru_cache(None)
def _sc_gather2(e_pad, d, nsub):
    C = 256
    epw = e_pad // _NW
    nch = epw // C
    rpw = epw // 128
    ngrp = d // _LN

    def body(ta, tb, ia2, ib2, z, ia_all, ib_all, ra, rb, zo, gsem, osem):
        wid = lax.axis_index("s") * _NC + lax.axis_index("c")
        irow0 = wid * rpw
        pltpu.sync_copy(ia2.at[pl.ds(irow0, rpw)], ia_all)
        pltpu.sync_copy(ib2.at[pl.ds(irow0, rpw)], ib_all)
        base0 = wid * epw

        def gather(g):
            for h in (0, 1):
                sl = pl.ds(h * 128, 128)
                pltpu.async_copy(ta.at[ia_all.at[2 * g + h]], ra.at[sl], gsem)
                pltpu.async_copy(tb.at[ib_all.at[2 * g + h]], rb.at[sl], gsem)

        def wait_gather():
            for _ in range(4):
                pltpu.make_async_copy(z.at[pl.ds(0, 128)],
                                      ra.at[pl.ds(0, 128)], gsem).wait()

        def wait_out():
            pltpu.make_async_copy(z.at[pl.ds(0, C)], zo, osem).wait()

        gather(0)

        @pl.loop(0, nch)
        def _chunk(g):
            wait_gather()

            @pl.when(g >= 1)
            def _wo():
                wait_out()

            @pl.loop(0, C)
            def _row(r):
                for j in range(ngrp):
                    sl = pl.ds(j * _LN, _LN)
                    if j < ngrp - nsub:
                        zv = ra[r, sl] + rb[r, sl]
                    else:
                        zv = ra[r, sl] - rb[r, sl]
                    zo[r, sl] = zv

            pltpu.async_copy(zo, z.at[pl.ds(base0 + g * C, C)], osem)

            @pl.when(g + 1 < nch)
            def _nx():
                gather(g + 1)

        wait_out()

    return pl.kernel(
        body,
        out_type=jax.ShapeDtypeStruct((e_pad, d), F32),
        mesh=_sc_mesh(),
        compiler_params=_sc_params(),
        scratch_types=[
            pltpu.VMEM((rpw, 128), I32),
            pltpu.VMEM((rpw, 128), I32),
            pltpu.VMEM((C, d), F32),
            pltpu.VMEM((C, d), F32),
            pltpu.VMEM((C, d), F32),
            pltpu.SemaphoreType.DMA,
            pltpu.SemaphoreType.DMA,
        ],
    )


@functools.lru_cache(None)
def _sc_gather1(e_pad, d):
    C = 256
    epw = e_pad // _NW
    nch = epw // C
    rpw = epw // 128

    def body(ta, ia2, z, ia_all, z0, z1, gs0, gs1, os0, os1):
        wid = lax.axis_index("s") * _NC + lax.axis_index("c")
        irow0 = wid * rpw
        pltpu.sync_copy(ia2.at[pl.ds(irow0, rpw)], ia_all)
        base0 = wid * epw
        z_v = [z0, z1]
        gsem = [gs0, gs1]
        osem = [os0, os1]

        def gather(b, g):
            for h in (0, 1):
                sl = pl.ds(h * 128, 128)
                pltpu.async_copy(ta.at[ia_all.at[2 * g + h]], z_v[b].at[sl],
                                 gsem[b])

        def wait_gather(b):
            for _ in range(2):
                pltpu.make_async_copy(z.at[pl.ds(0, 128)],
                                      z_v[b].at[pl.ds(0, 128)],
                                      gsem[b]).wait()

        def wait_out(b):
            pltpu.make_async_copy(z.at[pl.ds(0, C)], z_v[b], osem[b]).wait()

        gather(0, 0)

        @pl.loop(0, nch)
        def _chunk(g):
            b = lax.rem(g, 2)
            for bb in (0, 1):
                @pl.when(b == bb)
                def _go():
                    wait_gather(bb)

                    @pl.when(g + 1 < nch)
                    def _nx():
                        @pl.when(g >= 1)
                        def _wo():
                            wait_out(1 - bb)
                        gather(1 - bb, g + 1)

                    pltpu.async_copy(z_v[bb], z.at[pl.ds(base0 + g * C, C)],
                                     osem[bb])

        wait_out(0)
        wait_out(1)

    return pl.kernel(
        body,
        out_type=jax.ShapeDtypeStruct((e_pad, d), F32),
        mesh=_sc_mesh(),
        compiler_params=_sc_params(),
        scratch_types=[
            pltpu.VMEM((rpw, 128), I32),
            pltpu.VMEM((C, d), F32),
            pltpu.VMEM((C, d), F32),
            pltpu.SemaphoreType.DMA,
            pltpu.SemaphoreType.DMA,
            pltpu.SemaphoreType.DMA,
            pltpu.SemaphoreType.DMA,
        ],
    )


@functools.lru_cache(None)
def _sc_scatter_add(e_pad, d, n_pad):
    C = 128
    epw = e_pad // _NW
    nch = epw // C
    rpw = epw // 128
    rpt = n_pad // _NS

    def body(m, idx2, zrs, out, i0, i1, m0, m1, acc, f0, f1, s0, s1):
        c = lax.axis_index("c")
        s = lax.axis_index("s")
        wid = s * _NC + c
        row0 = s * rpt
        pltpu.sync_copy(zrs.at[pl.ds(row0, rpt)], acc.at[pl.ds(row0, rpt)])
        plsc.subcore_barrier()
        base0 = wid * epw
        irow0 = wid * rpw

        i_v = [i0, i1]
        m_v = [m0, m1]
        fsem = [f0, f1]
        ssem = [s0, s1]

        def fetch(b, g):
            pltpu.sync_copy(idx2.at[pl.ds(irow0 + g, 1)], i_v[b])
            pltpu.async_copy(m.at[pl.ds(base0 + g * C, C)], m_v[b], fsem[b])

        def wait_fetch(b):
            pltpu.make_async_copy(m.at[pl.ds(0, C)], m_v[b], fsem[b]).wait()

        def scatter(b):
            pltpu.async_copy(m_v[b], acc.at[i_v[b].at[0]], ssem[b], add=True)

        def wait_scatter(b):
            pltpu.make_async_copy(m_v[b], acc.at[i_v[b].at[0]],
                                  ssem[b]).wait()

        fetch(0, 0)

        @pl.loop(0, nch)
        def _chunk(g):
            b = lax.rem(g, 2)
            for bb in (0, 1):
                @pl.when(b == bb)
                def _go():
                    wait_fetch(bb)

                    @pl.when(g + 1 < nch)
                    def _nx():
                        @pl.when(g >= 1)
                        def _ws():
                            wait_scatter(1 - bb)
                        fetch(1 - bb, g + 1)

                    scatter(bb)

        wait_scatter(0)
        wait_scatter(1)
        plsc.subcore_barrier()
        pltpu.sync_copy(acc.at[pl.ds(row0, rpt)],
                        out.at[pl.ds(c * n_pad + row0, rpt)])

    return pl.kernel(
        body,
        out_type=jax.ShapeDtypeStruct((2 * n_pad, d), F32),
        mesh=_sc_mesh(),
        compiler_params=_sc_params(),
        scratch_types=[
            pltpu.VMEM((1, 128), I32),
            pltpu.VMEM((1, 128), I32),
            pltpu.VMEM((C, d), F32),
            pltpu.VMEM((C, d), F32),
            pltpu.VMEM_SHARED((n_pad, d), F32),
            pltpu.SemaphoreType.DMA,
            pltpu.SemaphoreType.DMA,
            pltpu.SemaphoreType.DMA,
            pltpu.SemaphoreType.DMA,
        ],
    )


@functools.lru_cache(None)
def _tc_linear_call(r, k, f, act, blk):
    def body(x_ref, w_ref, b_ref, o_ref):
        y = _dot(x_ref[...], w_ref[...]) + b_ref[...]
        o_ref[...] = _silu(y) if act else y

    return pl.pallas_call(
        body,
        grid=(r // blk,),
        in_specs=[
            pl.BlockSpec((blk, k), lambda i: (i, 0)),
            pl.BlockSpec((k, f), lambda i: (0, 0)),
            pl.BlockSpec((1, f), lambda i: (0, 0)),
        ],
        out_specs=pl.BlockSpec((blk, f), lambda i: (i, 0)),
        out_shape=jax.ShapeDtypeStruct((r, f), F32),
    )


def _tc_linear(x, w, b, act):
    r, k = x.shape
    f = w.shape[1]
    return _tc_linear_call(r, k, f, act, _blk_rows(r))(x, w, b.reshape(1, f))


@functools.lru_cache(None)
def _tc_msg_lig_call(e, blk):
    def body(z_ref, ea_ref, w_ref, o_ref):
        o_ref[...] = _silu(z_ref[...] + _dot(ea_ref[...], w_ref[...]))

    return pl.pallas_call(
        body,
        grid=(e // blk,),
        in_specs=[
            pl.BlockSpec((blk, 128), lambda i: (i, 0)),
            pl.BlockSpec((blk, 16), lambda i: (i, 0)),
            pl.BlockSpec((16, 128), lambda i: (0, 0)),
        ],
        out_specs=pl.BlockSpec((blk, 128), lambda i: (i, 0)),
        out_shape=jax.ShapeDtypeStruct((e, 128), F32),
    )


@functools.lru_cache(None)
def _tc_msg_cx_call(e, blk):
    def body(z_ref, ea_ref, we_ref, wd2_ref, wx_ref, bx_ref, o_ref):
        zz = z_ref[...]
        z128 = zz[:, :128]
        rel = zz[:, 128:131]
        d2 = jnp.sum(rel * rel, axis=1, keepdims=True)
        d2b = d2.astype(jnp.bfloat16).astype(F32)
        wd2 = wd2_ref[...].astype(jnp.bfloat16).astype(F32)
        mm = _silu(z128 + d2b * wd2 + _dot(ea_ref[...], we_ref[...]))
        coef = jnp.tanh(_dot(mm, wx_ref[...])[:, 0:1] + bx_ref[...][:, 0:1])
        relc = rel * coef
        ones = jnp.ones((blk, 1), F32)
        zer = jnp.zeros((blk, 12), F32)
        o_ref[...] = jnp.concatenate([mm, relc, ones, zer], axis=1)

    return pl.pallas_call(
        body,
        grid=(e // blk,),
        in_specs=[
            pl.BlockSpec((blk, 144), lambda i: (i, 0)),
            pl.BlockSpec((blk, 8), lambda i: (i, 0)),
            pl.BlockSpec((8, 128), lambda i: (0, 0)),
            pl.BlockSpec((1, 128), lambda i: (0, 0)),
            pl.BlockSpec((128, 128), lambda i: (0, 0)),
            pl.BlockSpec((1, 128), lambda i: (0, 0)),
        ],
        out_specs=pl.BlockSpec((blk, 144), lambda i: (i, 0)),
        out_shape=jax.ShapeDtypeStruct((e, 144), F32),
    )


@functools.lru_cache(None)
def _tc_msg_aa_call(e, blk):
    def body(z_ref, es_ref, ev_ref, we_ref, wv_ref, o_ref):
        ev = ev_ref[...][:, 0:3]
        evn = jnp.sqrt(jnp.sum(ev * ev, axis=1, keepdims=True) + 1e-8)
        evnb = evn.astype(jnp.bfloat16).astype(F32)
        wvb = wv_ref[...].astype(jnp.bfloat16).astype(F32)
        ma = _silu(z_ref[...] + _dot(es_ref[...], we_ref[...])
                   + evnb * wvb)
        ones = jnp.ones((blk, 1), F32)
        zer = jnp.zeros((blk, 15), F32)
        o_ref[...] = jnp.concatenate([ma, ones, zer], axis=1)

    return pl.pallas_call(
        body,
        grid=(e // blk,),
        in_specs=[
            pl.BlockSpec((blk, 128), lambda i: (i, 0)),
            pl.BlockSpec((blk, 32), lambda i: (i, 0)),
            pl.BlockSpec((blk, 8), lambda i: (i, 0)),
            pl.BlockSpec((32, 128), lambda i: (0, 0)),
            pl.BlockSpec((1, 128), lambda i: (0, 0)),
        ],
        out_specs=pl.BlockSpec((blk, 144), lambda i: (i, 0)),
        out_shape=jax.ShapeDtypeStruct((e, 144), F32),
    )


@functools.lru_cache(None)
def _tc_upd_call(n, blk):
    nb = n // blk

    def body(h_ref, s0_ref, s1_ref, wh_ref, wa_ref, b_ref, o_ref):
        agg = s0_ref[...] + s1_ref[...]
        o_ref[...] = _silu(_dot(h_ref[...], wh_ref[...])
                           + _dot(agg, wa_ref[...]) + b_ref[...])

    return pl.pallas_call(
        body,
        grid=(nb,),
        in_specs=[
            pl.BlockSpec((blk, 128), lambda i: (i, 0)),
            pl.BlockSpec((blk, 128), lambda i: (i, 0)),
            pl.BlockSpec((blk, 128), lambda i: (i + nb, 0)),
            pl.BlockSpec((128, 128), lambda i: (0, 0)),
            pl.BlockSpec((128, 128), lambda i: (0, 0)),
            pl.BlockSpec((1, 128), lambda i: (0, 0)),
        ],
        out_specs=pl.BlockSpec((blk, 128), lambda i: (i, 0)),
        out_shape=jax.ShapeDtypeStruct((n, 128), F32),
    )


@functools.lru_cache(None)
def _tc_upd_cx_call(n, blk):
    nb = n // blk

    def body(h_ref, p_ref, s0_ref, s1_ref, wh_ref, wa_ref, b_ref,
             oh_ref, op_ref):
        agg = s0_ref[...] + s1_ref[...]
        aggm = agg[:, :128]
        extras = agg[:, 128:144]
        cnt = extras[:, 3:4]
        inv = 1.0 / jnp.maximum(cnt, 1.0)
        lane = lax.broadcasted_iota(I32, (blk, 16), 1)
        op_ref[...] = p_ref[...] + jnp.where(lane < 3, extras * inv, 0.0)
        oh_ref[...] = _silu(_dot(h_ref[...], wh_ref[...])
                            + _dot(aggm, wa_ref[...]) + b_ref[...])

    return pl.pallas_call(
        body,
        grid=(nb,),
        in_specs=[
            pl.BlockSpec((blk, 128), lambda i: (i, 0)),
            pl.BlockSpec((blk, 16), lambda i: (i, 0)),
            pl.BlockSpec((blk, 144), lambda i: (i, 0)),
            pl.BlockSpec((blk, 144), lambda i: (i + nb, 0)),
            pl.BlockSpec((128, 128), lambda i: (0, 0)),
            pl.BlockSpec((128, 128), lambda i: (0, 0)),
            pl.BlockSpec((1, 128), lambda i: (0, 0)),
        ],
        out_specs=[
            pl.BlockSpec((blk, 128), lambda i: (i, 0)),
            pl.BlockSpec((blk, 16), lambda i: (i, 0)),
        ],
        out_shape=[
            jax.ShapeDtypeStruct((n, 128), F32),
            jax.ShapeDtypeStruct((n, 16), F32),
        ],
    )


@functools.lru_cache(None)
def _tc_upd_aa_call(n, blk):
    nb = n // blk

    def body(h_ref, s0_ref, s1_ref, wh_ref, wa_ref, b_ref, o_ref):
        agg = s0_ref[...] + s1_ref[...]
        cnt = agg[:, 128:129]
        aggmean = agg[:, :128] / jnp.maximum(cnt, 1.0)
        o_ref[...] = _silu(_dot(h_ref[...], wh_ref[...])
                           + _dot(aggmean, wa_ref[...]) + b_ref[...])

    return pl.pallas_call(
        body,
        grid=(nb,),
        in_specs=[
            pl.BlockSpec((blk, 128), lambda i: (i, 0)),
            pl.BlockSpec((blk, 144), lambda i: (i, 0)),
            pl.BlockSpec((blk, 144), lambda i: (i + nb, 0)),
            pl.BlockSpec((128, 128), lambda i: (0, 0)),
            pl.BlockSpec((128, 128), lambda i: (0, 0)),
            pl.BlockSpec((1, 128), lambda i: (0, 0)),
        ],
        out_specs=pl.BlockSpec((blk, 128), lambda i: (i, 0)),
        out_shape=jax.ShapeDtypeStruct((n, 128), F32),
    )


@functools.lru_cache(None)
def _tc_embed_aa_call(n, blk):
    def body(s8_ref, sq_ref, v16_ref, ws_ref, woh_ref, wv_ref, b_ref, o_ref):
        sq = sq_ref[...][:, 0:1]
        i20 = lax.broadcasted_iota(I32, (1, 20), 1).astype(F32)
        oh = (sq == i20).astype(F32)
        v = v16_ref[...]

        def nrm(k):
            sl = v[:, 3 * k:3 * k + 3]
            return jnp.sqrt(jnp.sum(sl * sl, axis=1, keepdims=True) + 1e-8)

        vn = jnp.concatenate([nrm(0), nrm(1), nrm(2)], axis=1)
        o_ref[...] = _silu(_dot(s8_ref[...], ws_ref[...])
                           + _dot(oh, woh_ref[...])
                           + _dot(vn, wv_ref[...]) + b_ref[...])

    return pl.pallas_call(
        body,
        grid=(n // blk,),
        in_specs=[
            pl.BlockSpec((blk, 8), lambda i: (i, 0)),
            pl.BlockSpec((blk, 8), lambda i: (i, 0)),
            pl.BlockSpec((blk, 16), lambda i: (i, 0)),
            pl.BlockSpec((8, 128), lambda i: (0, 0)),
            pl.BlockSpec((20, 128), lambda i: (0, 0)),
            pl.BlockSpec((3, 128), lambda i: (0, 0)),
            pl.BlockSpec((1, 128), lambda i: (0, 0)),
        ],
        out_specs=pl.BlockSpec((blk, 128), lambda i: (i, 0)),
        out_shape=jax.ShapeDtypeStruct((n, 128), F32),
    )


@functools.lru_cache(None)
def _tc_eea_call(e, blk):
    def body(es_ref, ev_ref, we_ref, wv_ref, o_ref):
        ev = ev_ref[...][:, 0:3]
        evn = jnp.sqrt(jnp.sum(ev * ev, axis=1, keepdims=True) + 1e-8)
        evnb = evn.astype(jnp.bfloat16).astype(F32)
        wvb = wv_ref[...].astype(jnp.bfloat16).astype(F32)
        o_ref[...] = _dot(es_ref[...], we_ref[...]) + evnb * wvb

    return pl.pallas_call(
        body,
        grid=(e // blk,),
        in_specs=[
            pl.BlockSpec((blk, 32), lambda i: (i, 0)),
            pl.BlockSpec((blk, 8), lambda i: (i, 0)),
            pl.BlockSpec((32, 128), lambda i: (0, 0)),
            pl.BlockSpec((1, 128), lambda i: (0, 0)),
        ],
        out_specs=pl.BlockSpec((blk, 128), lambda i: (i, 0)),
        out_shape=jax.ShapeDtypeStruct((e, 128), F32),
    )


@functools.lru_cache(None)
def _tc_pool_call(n, blk):
    def body(x_ref, bid_ref, s_ref, c_ref):
        @pl.when(pl.program_id(0) == 0)
        def _init():
            s_ref[...] = jnp.zeros_like(s_ref)
            c_ref[...] = jnp.zeros_like(c_ref)

        bid = bid_ref[...]
        i16 = lax.broadcasted_iota(I32, (16, blk), 0)
        one_t = (i16 == bid).astype(F32)
        s_ref[...] += lax.dot_general(one_t, x_ref[...],
                                      (((1,), (0,)), ((), ())),
                                      precision=lax.Precision.HIGHEST,
                                      preferred_element_type=F32)
        csum = jnp.sum(one_t, axis=1, keepdims=True)
        c_ref[...] += jnp.broadcast_to(csum, (16, 128))

    return pl.pallas_call(
        body,
        grid=(n // blk,),
        in_specs=[
            pl.BlockSpec((blk, 128), lambda i: (i, 0)),
            pl.BlockSpec((1, blk), lambda i: (0, i)),
        ],
        out_specs=[
            pl.BlockSpec((16, 128), lambda i: (0, 0)),
            pl.BlockSpec((16, 128), lambda i: (0, 0)),
        ],
        out_shape=[
            jax.ShapeDtypeStruct((16, 128), F32),
            jax.ShapeDtypeStruct((16, 128), F32),
        ],
    )


@functools.lru_cache(None)
def _tc_head_call():
    def body(sl_ref, cl_ref, sa_ref, ca_ref, sc_ref, cc_ref,
             wi_ref, bi_ref, wg1_ref, bg1_ref, wg2_ref, bg2_ref,
             o1_ref, o2_ref):
        p_l = sl_ref[...] / jnp.maximum(cl_ref[...], 1.0)
        p_a = sa_ref[...] / jnp.maximum(ca_ref[...], 1.0)
        x_c = sc_ref[...] / jnp.maximum(cc_ref[...], 1.0)
        inter = _silu(_dot(jnp.concatenate([p_l, p_a], axis=1), wi_ref[...])
                      + bi_ref[...])

        def g(v):
            h1 = jnp.maximum(_dot(v, wg1_ref[...]) + bg1_ref[...], 0.0)
            return _dot(h1, wg2_ref[...]) + bg2_ref[...]

        o1_ref[...] = g(inter)
        o2_ref[...] = g(x_c)

    return pl.pallas_call(
        body,
        out_shape=[
            jax.ShapeDtypeStruct((16, 128), F32),
            jax.ShapeDtypeStruct((16, 128), F32),
        ],
    )


def kernel(x_l, edge_attr_l, x_c, pos_c, edge_attr_c, node_s, node_v,
           edge_s, edge_v, params, edge_index_l, batch_l, edge_index_c,
           batch_c, seq, edge_index_aa, batch_aa):
    p = params
    n_l, n_c, n_a = x_l.shape[0], x_c.shape[0], node_s.shape[0]
    e_l, e_c, e_a = (edge_index_l.shape[1], edge_index_c.shape[1],
                     edge_index_aa.shape[1])
    np_l, np_c, np_a = (_ceil_to(n_l, 2048), _ceil_to(n_c, 2048),
                        _ceil_to(n_a, 2048))
    ep_l, ep_c, ep_a = (_ceil_to(e_l, 4096), _ceil_to(e_c, 4096),
                        _ceil_to(e_a, 4096))

    srcl_g = _ipad(edge_index_l[0], ep_l, 0)
    dstl_g = _ipad(edge_index_l[1], ep_l, 0)
    dstl_s = _ipad(edge_index_l[1], ep_l, n_l)
    srcc_g = _ipad(edge_index_c[0], ep_c, 0)
    dstc_g = _ipad(edge_index_c[1], ep_c, 0)
    dstc_s = _ipad(edge_index_c[1], ep_c, n_c)
    saa_g = _ipad(edge_index_aa[0], ep_a, 0)
    daa_s = _ipad(edge_index_aa[1], ep_a, n_a)

    ea_l = _rpad(edge_attr_l, ep_l)
    ea_c = _cpad(_rpad(edge_attr_c, ep_c), 8)
    es_a = _rpad(edge_s, ep_a)
    ev_a = _cpad(_rpad(edge_v.reshape(e_a, 3), ep_a), 8)

    bid_l = _ipad(batch_l, np_l, 63).reshape(1, np_l)
    bid_c = _ipad(batch_c, np_c, 63).reshape(1, np_c)
    bid_a = _ipad(batch_aa, np_a, 63).reshape(1, np_a)

    zeros_l = jnp.zeros((np_l, 128), F32)
    zeros_c = jnp.zeros((np_c, 144), F32)
    zeros_a = jnp.zeros((np_a, 144), F32)

    def dep(idx_arr, t):
        if t is None:
            return idx_arr
        out, _ = lax.optimization_barrier((idx_arr, t))
        return out

    tok = None

    dstl_g2 = dstl_g.reshape(-1, 128)
    srcl_g2 = srcl_g.reshape(-1, 128)
    dstl_s2 = dstl_s.reshape(-1, 128)
    dstc_g2 = dstc_g.reshape(-1, 128)
    srcc_g2 = srcc_g.reshape(-1, 128)
    dstc_s2 = dstc_s.reshape(-1, 128)
    saa_g2 = saa_g.reshape(-1, 128)
    daa_s2 = daa_s.reshape(-1, 128)

    h = _tc_linear(_rpad(x_l, np_l), p['W_lig'], p['b_lig'], True)
    hc = _tc_linear(_rpad(x_c, np_c), p['W_cx'], p['b_cx'], True)
    pos16 = _cpad(_rpad(pos_c, np_c), 16)
    ns8 = _cpad(_rpad(node_s, np_a), 8)
    sq8 = _cpad(_rpad(seq.astype(F32).reshape(n_a, 1), np_a), 8)
    v16 = _cpad(_rpad(node_v.reshape(n_a, 9), np_a), 16)
    w_aa = p['W_aa']
    ws8 = jnp.pad(w_aa[:6], ((0, 2), (0, 0)))
    ha = _tc_embed_aa_call(np_a, 2048)(ns8, sq8, v16, ws8, w_aa[6:26],
                                       w_aa[26:29],
                                       p['b_aa'].reshape(1, 128))

    for i in ('1', '2', '3'):
        wm = p['Wm' + i]
        w2 = jnp.concatenate([wm[:128], wm[128:256]], axis=1)
        b2 = jnp.concatenate([p['bm' + i], jnp.zeros((128,), F32)])
        ab = _tc_linear(h, w2, b2, False)
        z = _sc_gather2(ep_l, 128, 0)(ab[:, :128], ab[:, 128:],
                                      dstl_g2, srcl_g2)
        m = _tc_msg_lig_call(ep_l, 2048)(z, ea_l, wm[256:])
        s = _sc_scatter_add(ep_l, 128, np_l)(m, dep(dstl_s2, tok), zeros_l)
        tok = s[0, 0]
        wu = p['Wu' + i]
        h = _tc_upd_call(np_l, 2048)(h, s, s, wu[:128], wu[128:],
                                     p['bu' + i].reshape(1, 128))
    s_l, c_l = _tc_pool_call(np_l, 2048)(h, bid_l)

    for i in ('1', '2', '3'):
        we = p['We' + i]
        w2 = jnp.concatenate([we[:128], we[128:256]], axis=1)
        b2 = jnp.concatenate([p['be' + i], jnp.zeros((128,), F32)])
        ab = _tc_linear(hc, w2, b2, False)
        ta = jnp.concatenate([ab[:, :128], pos16], axis=1)
        tb = jnp.concatenate([ab[:, 128:], pos16], axis=1)
        z = _sc_gather2(ep_c, 144, 1)(ta, tb, dstc_g2, srcc_g2)
        we_e = jnp.pad(we[257:261], ((0, 4), (0, 0)))
        wx = jnp.pad(p['Wx' + i], ((0, 0), (0, 127)))
        bx = jnp.broadcast_to(p['bx' + i].reshape(1, 1), (1, 128))
        m = _tc_msg_cx_call(ep_c, 2048)(z, ea_c, we_e, we[256:257], wx, bx)
        s = _sc_scatter_add(ep_c, 144, np_c)(m, dep(dstc_s2, tok), zeros_c)
        tok = s[0, 0]
        wh = p['Wh' + i]
        hc, pos16 = _tc_upd_cx_call(np_c, 2048)(
            hc, pos16, s, s, wh[:128], wh[128:],
            p['bh' + i].reshape(1, 128))
    s_c, c_c = _tc_pool_call(np_c, 2048)(hc, bid_c)

    aa_t = _tc_linear(ha, p['W_amsg'][:128], p['b_amsg'], False)
    z = _sc_gather1(ep_a, 128)(aa_t, saa_g2)
    m = _tc_msg_aa_call(ep_a, 2048)(z, es_a, ev_a, p['W_amsg'][128:160],
                                    p['W_amsg'][160:161])
    s = _sc_scatter_add(ep_a, 144, np_a)(m, dep(daa_s2, tok), zeros_a)
    wv = p['W_aupd']
    ha = _tc_upd_aa_call(np_a, 2048)(ha, s, s, wv[:128], wv[128:],
                                     p['b_aupd'].reshape(1, 128))
    s_a, c_a = _tc_pool_call(np_a, 2048)(ha, bid_a)

    wg2 = jnp.pad(p['W_g2'], ((0, 0), (0, 127)))
    bg2 = jnp.broadcast_to(p['b_g2'].reshape(1, 1), (1, 128))
    o1, o2 = _tc_head_call()(s_l, c_l, s_a, c_a, s_c, c_c,
                             p['W_int'], p['b_int'].reshape(1, 128),
                             p['W_g1'], p['b_g1'].reshape(1, 256),
                             wg2, bg2)
    return o1[:, :1], o2[:, :1]


# speedup vs baseline: 1.0773x; 1.0013x over previous
import dataclasses
import functools

import jax
import jax.numpy as jnp
from jax import lax
from jax.experimental import pallas as pl
from jax.experimental.pallas import tpu as pltpu
from jax.experimental.pallas import tpu_sc as plsc

F32 = jnp.float32
I32 = jnp.int32
_NC, _NS, _LN = 2, 16, 16
_NW = _NC * _NS
_CHUNK = 128
_PREC = lax.Precision.DEFAULT


def _silu(x):
    return x * jax.nn.sigmoid(x)


def _dot(a, b):
    return lax.dot_general(a, b, (((1,), (0,)), ((), ())),
                           precision=_PREC, preferred_element_type=F32)


def _ceil_to(n, m):
    return -(-n // m) * m


def _rpad(x, rows):
    return jnp.pad(x, ((0, rows - x.shape[0]),) + ((0, 0),) * (x.ndim - 1))


def _cpad(x, cols):
    return jnp.pad(x, ((0, 0), (0, cols - x.shape[1])))


def _ipad(x, n, val):
    return jnp.concatenate([x.astype(I32), jnp.full((n - x.shape[0],), val, I32)])


def _blk_rows(r):
    return 2048 if r % 2048 == 0 else (1024 if r % 1024 == 0 else r)


def _sc_params():
    cp = pltpu.CompilerParams()
    if "use_tc_tiling_on_sc" in pltpu.CompilerParams.__dataclass_fields__:
        cp = dataclasses.replace(cp, use_tc_tiling_on_sc=False)
    return cp


def _sc_mesh():
    return plsc.VectorSubcoreMesh(core_axis_name="c", subcore_axis_name="s",
                                  num_cores=_NC, num_subcores=_NS)


@functools.lru_cache(None)
def _sc_gather2(e_pad, d, nsub):
    C = 256
    epw = e_pad // _NW
    nch = epw // C
    rpw = epw // 128
    ngrp = d // _LN

    def body(ta, tb, ia2, ib2, z, ia_all, ib_all, ra, rb, zo, gsem, osem):
        wid = lax.axis_index("s") * _NC + lax.axis_index("c")
        irow0 = wid * rpw
        pltpu.sync_copy(ia2.at[pl.ds(irow0, rpw)], ia_all)
        pltpu.sync_copy(ib2.at[pl.ds(irow0, rpw)], ib_all)
        base0 = wid * epw

        def gather(g):
            for h in (0, 1):
                sl = pl.ds(h * 128, 128)
                pltpu.async_copy(ta.at[ia_all.at[2 * g + h]], ra.at[sl], gsem)
                pltpu.async_copy(tb.at[ib_all.at[2 * g + h]], rb.at[sl], gsem)

        def wait_gather():
            for _ in range(4):
                pltpu.make_async_copy(z.at[pl.ds(0, 128)],
                                      ra.at[pl.ds(0, 128)], gsem).wait()

        def wait_out():
            pltpu.make_async_copy(z.at[pl.ds(0, C)], zo, osem).wait()

        gather(0)

        @pl.loop(0, nch)
        def _chunk(g):
            wait_gather()

            @pl.when(g >= 1)
            def _wo():
                wait_out()

            @pl.loop(0, C)
            def _row(r):
                for j in range(ngrp):
                    sl = pl.ds(j * _LN, _LN)
                    if j < ngrp - nsub:
                        zv = ra[r, sl] + rb[r, sl]
                    else:
                        zv = ra[r, sl] - rb[r, sl]
                    zo[r, sl] = zv

            pltpu.async_copy(zo, z.at[pl.ds(base0 + g * C, C)], osem)

            @pl.when(g + 1 < nch)
            def _nx():
                gather(g + 1)

        wait_out()

    return pl.kernel(
        body,
        out_type=jax.ShapeDtypeStruct((e_pad, d), F32),
        mesh=_sc_mesh(),
        compiler_params=_sc_params(),
        scratch_types=[
            pltpu.VMEM((rpw, 128), I32),
            pltpu.VMEM((rpw, 128), I32),
            pltpu.VMEM((C, d), F32),
            pltpu.VMEM((C, d), F32),
            pltpu.VMEM((C, d), F32),
            pltpu.SemaphoreType.DMA,
            pltpu.SemaphoreType.DMA,
        ],
    )


@functools.lru_cache(None)
def _sc_gather1(e_pad, d):
    C = 256
    epw = e_pad // _NW
    nch = epw // C
    rpw = epw // 128

    def body(ta, ia2, z, ia_all, z0, z1, gs0, gs1, os0, os1):
        wid = lax.axis_index("s") * _NC + lax.axis_index("c")
        irow0 = wid * rpw
        pltpu.sync_copy(ia2.at[pl.ds(irow0, rpw)], ia_all)
        base0 = wid * epw
        z_v = [z0, z1]
        gsem = [gs0, gs1]
        osem = [os0, os1]

        def gather(b, g):
            for h in (0, 1):
                sl = pl.ds(h * 128, 128)
                pltpu.async_copy(ta.at[ia_all.at[2 * g + h]], z_v[b].at[sl],
                                 gsem[b])

        def wait_gather(b):
            for _ in range(2):
                pltpu.make_async_copy(z.at[pl.ds(0, 128)],
                                      z_v[b].at[pl.ds(0, 128)],
                                      gsem[b]).wait()

        def wait_out(b):
            pltpu.make_async_copy(z.at[pl.ds(0, C)], z_v[b], osem[b]).wait()

        gather(0, 0)

        @pl.loop(0, nch)
        def _chunk(g):
            b = lax.rem(g, 2)
            for bb in (0, 1):
                @pl.when(b == bb)
                def _go():
                    wait_gather(bb)

                    @pl.when(g + 1 < nch)
                    def _nx():
                        @pl.when(g >= 1)
                        def _wo():
                            wait_out(1 - bb)
                        gather(1 - bb, g + 1)

                    pltpu.async_copy(z_v[bb], z.at[pl.ds(base0 + g * C, C)],
                                     osem[bb])

        wait_out(0)
        wait_out(1)

    return pl.kernel(
        body,
        out_type=jax.ShapeDtypeStruct((e_pad, d), F32),
        mesh=_sc_mesh(),
        compiler_params=_sc_params(),
        scratch_types=[
            pltpu.VMEM((rpw, 128), I32),
            pltpu.VMEM((C, d), F32),
            pltpu.VMEM((C, d), F32),
            pltpu.SemaphoreType.DMA,
            pltpu.SemaphoreType.DMA,
            pltpu.SemaphoreType.DMA,
            pltpu.SemaphoreType.DMA,
        ],
    )


@functools.lru_cache(None)
def _sc_scatter_add(e_pad, d, n_pad):
    C = 128
    epw = e_pad // _NW
    nch = epw // C
    rpw = epw // 128
    rpt = n_pad // _NS

    def body(m, idx2, zrs, out, i0, i1, m0, m1, acc, f0, f1, s0, s1):
        c = lax.axis_index("c")
        s = lax.axis_index("s")
        wid = s * _NC + c
        row0 = s * rpt
        pltpu.sync_copy(zrs.at[pl.ds(row0, rpt)], acc.at[pl.ds(row0, rpt)])
        plsc.subcore_barrier()
        base0 = wid * epw
        irow0 = wid * rpw

        i_v = [i0, i1]
        m_v = [m0, m1]
        fsem = [f0, f1]
        ssem = [s0, s1]

        def fetch(b, g):
            pltpu.sync_copy(idx2.at[pl.ds(irow0 + g, 1)], i_v[b])
            pltpu.async_copy(m.at[pl.ds(base0 + g * C, C)], m_v[b], fsem[b])

        def wait_fetch(b):
            pltpu.make_async_copy(m.at[pl.ds(0, C)], m_v[b], fsem[b]).wait()

        def scatter(b):
            pltpu.async_copy(m_v[b], acc.at[i_v[b].at[0]], ssem[b], add=True)

        def wait_scatter(b):
            pltpu.make_async_copy(m_v[b], acc.at[i_v[b].at[0]],
                                  ssem[b]).wait()

        fetch(0, 0)

        @pl.loop(0, nch)
        def _chunk(g):
            b = lax.rem(g, 2)
            for bb in (0, 1):
                @pl.when(b == bb)
                def _go():
                    wait_fetch(bb)

                    @pl.when(g + 1 < nch)
                    def _nx():
                        @pl.when(g >= 1)
                        def _ws():
                            wait_scatter(1 - bb)
                        fetch(1 - bb, g + 1)

                    scatter(bb)

        wait_scatter(0)
        wait_scatter(1)
        plsc.subcore_barrier()
        pltpu.sync_copy(acc.at[pl.ds(row0, rpt)],
                        out.at[pl.ds(c * n_pad + row0, rpt)])

    return pl.kernel(
        body,
        out_type=jax.ShapeDtypeStruct((2 * n_pad, d), F32),
        mesh=_sc_mesh(),
        compiler_params=_sc_params(),
        scratch_types=[
            pltpu.VMEM((1, 128), I32),
            pltpu.VMEM((1, 128), I32),
            pltpu.VMEM((C, d), F32),
            pltpu.VMEM((C, d), F32),
            pltpu.VMEM_SHARED((n_pad, d), F32),
            pltpu.SemaphoreType.DMA,
            pltpu.SemaphoreType.DMA,
            pltpu.SemaphoreType.DMA,
            pltpu.SemaphoreType.DMA,
        ],
    )


@functools.lru_cache(None)
def _tc_linear_call(r, k, f, act, blk):
    def body(x_ref, w_ref, b_ref, o_ref):
        y = _dot(x_ref[...], w_ref[...]) + b_ref[...]
        o_ref[...] = _silu(y) if act else y

    return pl.pallas_call(
        body,
        grid=(r // blk,),
        in_specs=[
            pl.BlockSpec((blk, k), lambda i: (i, 0)),
            pl.BlockSpec((k, f), lambda i: (0, 0)),
            pl.BlockSpec((1, f), lambda i: (0, 0)),
        ],
        out_specs=pl.BlockSpec((blk, f), lambda i: (i, 0)),
        out_shape=jax.ShapeDtypeStruct((r, f), F32),
    )


def _tc_linear(x, w, b, act):
    r, k = x.shape
    f = w.shape[1]
    return _tc_linear_call(r, k, f, act, _blk_rows(r))(x, w, b.reshape(1, f))


@functools.lru_cache(None)
def _tc_msg_lig_call(e, blk):
    def body(z_ref, ea_ref, w_ref, o_ref):
        o_ref[...] = _silu(z_ref[...] + _dot(ea_ref[...], w_ref[...]))

    return pl.pallas_call(
        body,
        grid=(e // blk,),
        in_specs=[
            pl.BlockSpec((blk, 128), lambda i: (i, 0)),
            pl.BlockSpec((blk, 16), lambda i: (i, 0)),
            pl.BlockSpec((16, 128), lambda i: (0, 0)),
        ],
        out_specs=pl.BlockSpec((blk, 128), lambda i: (i, 0)),
        out_shape=jax.ShapeDtypeStruct((e, 128), F32),
    )


@functools.lru_cache(None)
def _tc_msg_cx_call(e, blk):
    def body(z_ref, ea_ref, we_ref, wd2_ref, wx_ref, bx_ref, o_ref):
        zz = z_ref[...]
        z128 = zz[:, :128]
        rel = zz[:, 128:131]
        d2 = jnp.sum(rel * rel, axis=1, keepdims=True)
        d2b = d2.astype(jnp.bfloat16).astype(F32)
        wd2 = wd2_ref[...].astype(jnp.bfloat16).astype(F32)
        mm = _silu(z128 + d2b * wd2 + _dot(ea_ref[...], we_ref[...]))
        coef = jnp.tanh(_dot(mm, wx_ref[...])[:, 0:1] + bx_ref[...][:, 0:1])
        relc = rel * coef
        ones = jnp.ones((blk, 1), F32)
        zer = jnp.zeros((blk, 12), F32)
        o_ref[...] = jnp.concatenate([mm, relc, ones, zer], axis=1)

    return pl.pallas_call(
        body,
        grid=(e // blk,),
        in_specs=[
            pl.BlockSpec((blk, 144), lambda i: (i, 0)),
            pl.BlockSpec((blk, 8), lambda i: (i, 0)),
            pl.BlockSpec((8, 128), lambda i: (0, 0)),
            pl.BlockSpec((1, 128), lambda i: (0, 0)),
            pl.BlockSpec((128, 128), lambda i: (0, 0)),
            pl.BlockSpec((1, 128), lambda i: (0, 0)),
        ],
        out_specs=pl.BlockSpec((blk, 144), lambda i: (i, 0)),
        out_shape=jax.ShapeDtypeStruct((e, 144), F32),
    )


@functools.lru_cache(None)
def _tc_msg_aa_call(e, blk):
    def body(z_ref, es_ref, ev_ref, we_ref, wv_ref, o_ref):
        ev = ev_ref[...][:, 0:3]
        evn = jnp.sqrt(jnp.sum(ev * ev, axis=1, keepdims=True) + 1e-8)
        evnb = evn.astype(jnp.bfloat16).astype(F32)
        wvb = wv_ref[...].astype(jnp.bfloat16).astype(F32)
        ma = _silu(z_ref[...] + _dot(es_ref[...], we_ref[...])
                   + evnb * wvb)
        ones = jnp.ones((blk, 1), F32)
        zer = jnp.zeros((blk, 15), F32)
        o_ref[...] = jnp.concatenate([ma, ones, zer], axis=1)

    return pl.pallas_call(
        body,
        grid=(e // blk,),
        in_specs=[
            pl.BlockSpec((blk, 128), lambda i: (i, 0)),
            pl.BlockSpec((blk, 32), lambda i: (i, 0)),
            pl.BlockSpec((blk, 8), lambda i: (i, 0)),
            pl.BlockSpec((32, 128), lambda i: (0, 0)),
            pl.BlockSpec((1, 128), lambda i: (0, 0)),
        ],
        out_specs=pl.BlockSpec((blk, 144), lambda i: (i, 0)),
        out_shape=jax.ShapeDtypeStruct((e, 144), F32),
    )


@functools.lru_cache(None)
def _tc_upd_call(n, blk):
    nb = n // blk

    def body(h_ref, s0_ref, s1_ref, wh_ref, wa_ref, b_ref, o_ref):
        agg = s0_ref[...] + s1_ref[...]
        o_ref[...] = _silu(_dot(h_ref[...], wh_ref[...])
                           + _dot(agg, wa_ref[...]) + b_ref[...])

    return pl.pallas_call(
        body,
        grid=(nb,),
        in_specs=[
            pl.BlockSpec((blk, 128), lambda i: (i, 0)),
            pl.BlockSpec((blk, 128), lambda i: (i, 0)),
            pl.BlockSpec((blk, 128), lambda i: (i + nb, 0)),
            pl.BlockSpec((128, 128), lambda i: (0, 0)),
            pl.BlockSpec((128, 128), lambda i: (0, 0)),
            pl.BlockSpec((1, 128), lambda i: (0, 0)),
        ],
        out_specs=pl.BlockSpec((blk, 128), lambda i: (i, 0)),
        out_shape=jax.ShapeDtypeStruct((n, 128), F32),
    )


@functools.lru_cache(None)
def _tc_upd_cx_call(n, blk):
    nb = n // blk

    def body(h_ref, p_ref, s0_ref, s1_ref, wh_ref, wa_ref, b_ref,
             oh_ref, op_ref):
        agg = s0_ref[...] + s1_ref[...]
        aggm = agg[:, :128]
        extras = agg[:, 128:144]
        cnt = extras[:, 3:4]
        inv = 1.0 / jnp.maximum(cnt, 1.0)
        lane = lax.broadcasted_iota(I32, (blk, 16), 1)
        op_ref[...] = p_ref[...] + jnp.where(lane < 3, extras * inv, 0.0)
        oh_ref[...] = _silu(_dot(h_ref[...], wh_ref[...])
                            + _dot(aggm, wa_ref[...]) + b_ref[...])

    return pl.pallas_call(
        body,
        grid=(nb,),
        in_specs=[
            pl.BlockSpec((blk, 128), lambda i: (i, 0)),
            pl.BlockSpec((blk, 16), lambda i: (i, 0)),
            pl.BlockSpec((blk, 144), lambda i: (i, 0)),
            pl.BlockSpec((blk, 144), lambda i: (i + nb, 0)),
            pl.BlockSpec((128, 128), lambda i: (0, 0)),
            pl.BlockSpec((128, 128), lambda i: (0, 0)),
            pl.BlockSpec((1, 128), lambda i: (0, 0)),
        ],
        out_specs=[
            pl.BlockSpec((blk, 128), lambda i: (i, 0)),
            pl.BlockSpec((blk, 16), lambda i: (i, 0)),
        ],
        out_shape=[
            jax.ShapeDtypeStruct((n, 128), F32),
            jax.ShapeDtypeStruct((n, 16), F32),
        ],
    )


@functools.lru_cache(None)
def _tc_upd_aa_call(n, blk):
    nb = n // blk

    def body(h_ref, s0_ref, s1_ref, wh_ref, wa_ref, b_ref, o_ref):
        agg = s0_ref[...] + s1_ref[...]
        cnt = agg[:, 128:129]
        aggmean = agg[:, :128] / jnp.maximum(cnt, 1.0)
        o_ref[...] = _silu(_dot(h_ref[...], wh_ref[...])
                           + _dot(aggmean, wa_ref[...]) + b_ref[...])

    return pl.pallas_call(
        body,
        grid=(nb,),
        in_specs=[
            pl.BlockSpec((blk, 128), lambda i: (i, 0)),
            pl.BlockSpec((blk, 144), lambda i: (i, 0)),
            pl.BlockSpec((blk, 144), lambda i: (i + nb, 0)),
            pl.BlockSpec((128, 128), lambda i: (0, 0)),
            pl.BlockSpec((128, 128), lambda i: (0, 0)),
            pl.BlockSpec((1, 128), lambda i: (0, 0)),
        ],
        out_specs=pl.BlockSpec((blk, 128), lambda i: (i, 0)),
        out_shape=jax.ShapeDtypeStruct((n, 128), F32),
    )


@functools.lru_cache(None)
def _tc_embed_aa_call(n, blk):
    def body(s8_ref, sq_ref, v16_ref, ws_ref, woh_ref, wv_ref, b_ref, o_ref):
        sq = sq_ref[...][:, 0:1]
        i20 = lax.broadcasted_iota(I32, (1, 20), 1).astype(F32)
        oh = (sq == i20).astype(F32)
        v = v16_ref[...]

        def nrm(k):
            sl = v[:, 3 * k:3 * k + 3]
            return jnp.sqrt(jnp.sum(sl * sl, axis=1, keepdims=True) + 1e-8)

        vn = jnp.concatenate([nrm(0), nrm(1), nrm(2)], axis=1)
        o_ref[...] = _silu(_dot(s8_ref[...], ws_ref[...])
                           + _dot(oh, woh_ref[...])
                           + _dot(vn, wv_ref[...]) + b_ref[...])

    return pl.pallas_call(
        body,
        grid=(n // blk,),
        in_specs=[
            pl.BlockSpec((blk, 8), lambda i: (i, 0)),
            pl.BlockSpec((blk, 8), lambda i: (i, 0)),
            pl.BlockSpec((blk, 16), lambda i: (i, 0)),
            pl.BlockSpec((8, 128), lambda i: (0, 0)),
            pl.BlockSpec((20, 128), lambda i: (0, 0)),
            pl.BlockSpec((3, 128), lambda i: (0, 0)),
            pl.BlockSpec((1, 128), lambda i: (0, 0)),
        ],
        out_specs=pl.BlockSpec((blk, 128), lambda i: (i, 0)),
        out_shape=jax.ShapeDtypeStruct((n, 128), F32),
    )


@functools.lru_cache(None)
def _tc_eea_call(e, blk):
    def body(es_ref, ev_ref, we_ref, wv_ref, o_ref):
        ev = ev_ref[...][:, 0:3]
        evn = jnp.sqrt(jnp.sum(ev * ev, axis=1, keepdims=True) + 1e-8)
        evnb = evn.astype(jnp.bfloat16).astype(F32)
        wvb = wv_ref[...].astype(jnp.bfloat16).astype(F32)
        o_ref[...] = _dot(es_ref[...], we_ref[...]) + evnb * wvb

    return pl.pallas_call(
        body,
        grid=(e // blk,),
        in_specs=[
            pl.BlockSpec((blk, 32), lambda i: (i, 0)),
            pl.BlockSpec((blk, 8), lambda i: (i, 0)),
            pl.BlockSpec((32, 128), lambda i: (0, 0)),
            pl.BlockSpec((1, 128), lambda i: (0, 0)),
        ],
        out_specs=pl.BlockSpec((blk, 128), lambda i: (i, 0)),
        out_shape=jax.ShapeDtypeStruct((e, 128), F32),
    )


@functools.lru_cache(None)
def _tc_pool_call(n, blk):
    def body(x_ref, bid_ref, s_ref, c_ref):
        @pl.when(pl.program_id(0) == 0)
        def _init():
            s_ref[...] = jnp.zeros_like(s_ref)
            c_ref[...] = jnp.zeros_like(c_ref)

        bid = bid_ref[...]
        i16 = lax.broadcasted_iota(I32, (16, blk), 0)
        one_t = (i16 == bid).astype(F32)
        s_ref[...] += lax.dot_general(one_t, x_ref[...],
                                      (((1,), (0,)), ((), ())),
                                      precision=lax.Precision.HIGHEST,
                                      preferred_element_type=F32)
        csum = jnp.sum(one_t, axis=1, keepdims=True)
        c_ref[...] += jnp.broadcast_to(csum, (16, 128))

    return pl.pallas_call(
        body,
        grid=(n // blk,),
        in_specs=[
            pl.BlockSpec((blk, 128), lambda i: (i, 0)),
            pl.BlockSpec((1, blk), lambda i: (0, i)),
        ],
        out_specs=[
            pl.BlockSpec((16, 128), lambda i: (0, 0)),
            pl.BlockSpec((16, 128), lambda i: (0, 0)),
        ],
        out_shape=[
            jax.ShapeDtypeStruct((16, 128), F32),
            jax.ShapeDtypeStruct((16, 128), F32),
        ],
    )


@functools.lru_cache(None)
def _tc_head_call():
    def body(sl_ref, cl_ref, sa_ref, ca_ref, sc_ref, cc_ref,
             wi_ref, bi_ref, wg1_ref, bg1_ref, wg2_ref, bg2_ref,
             o1_ref, o2_ref):
        p_l = sl_ref[...] / jnp.maximum(cl_ref[...], 1.0)
        p_a = sa_ref[...] / jnp.maximum(ca_ref[...], 1.0)
        x_c = sc_ref[...] / jnp.maximum(cc_ref[...], 1.0)
        inter = _silu(_dot(jnp.concatenate([p_l, p_a], axis=1), wi_ref[...])
                      + bi_ref[...])

        def g(v):
            h1 = jnp.maximum(_dot(v, wg1_ref[...]) + bg1_ref[...], 0.0)
            return _dot(h1, wg2_ref[...]) + bg2_ref[...]

        o1_ref[...] = g(inter)
        o2_ref[...] = g(x_c)

    return pl.pallas_call(
        body,
        out_shape=[
            jax.ShapeDtypeStruct((16, 128), F32),
            jax.ShapeDtypeStruct((16, 128), F32),
        ],
    )


def kernel(x_l, edge_attr_l, x_c, pos_c, edge_attr_c, node_s, node_v,
           edge_s, edge_v, params, edge_index_l, batch_l, edge_index_c,
           batch_c, seq, edge_index_aa, batch_aa):
    p = params
    n_l, n_c, n_a = x_l.shape[0], x_c.shape[0], node_s.shape[0]
    e_l, e_c, e_a = (edge_index_l.shape[1], edge_index_c.shape[1],
                     edge_index_aa.shape[1])
    np_l, np_c, np_a = (_ceil_to(n_l, 2048), _ceil_to(n_c, 2048),
                        _ceil_to(n_a, 2048))
    ep_l, ep_c, ep_a = (_ceil_to(e_l, 4096), _ceil_to(e_c, 4096),
                        _ceil_to(e_a, 4096))

    srcl_g = _ipad(edge_index_l[0], ep_l, 0)
    dstl_g = _ipad(edge_index_l[1], ep_l, 0)
    dstl_s = _ipad(edge_index_l[1], ep_l, n_l)
    srcc_g = _ipad(edge_index_c[0], ep_c, 0)
    dstc_g = _ipad(edge_index_c[1], ep_c, 0)
    dstc_s = _ipad(edge_index_c[1], ep_c, n_c)
    saa_g = _ipad(edge_index_aa[0], ep_a, 0)
    daa_s = _ipad(edge_index_aa[1], ep_a, n_a)

    ea_l = _rpad(edge_attr_l, ep_l)
    ea_c = _cpad(_rpad(edge_attr_c, ep_c), 8)
    es_a = _rpad(edge_s, ep_a)
    ev_a = _cpad(_rpad(edge_v.reshape(e_a, 3), ep_a), 8)

    bid_l = _ipad(batch_l, np_l, 63).reshape(1, np_l)
    bid_c = _ipad(batch_c, np_c, 63).reshape(1, np_c)
    bid_a = _ipad(batch_aa, np_a, 63).reshape(1, np_a)

    zeros_l = jnp.zeros((np_l, 128), F32)
    zeros_c = jnp.zeros((np_c, 144), F32)
    zeros_a = jnp.zeros((np_a, 144), F32)

    def dep(idx_arr, t):
        if t is None:
            return idx_arr
        out, _ = lax.optimization_barrier((idx_arr, t))
        return out

    tok = None

    dstl_g2 = dstl_g.reshape(-1, 128)
    srcl_g2 = srcl_g.reshape(-1, 128)
    dstl_s2 = dstl_s.reshape(-1, 128)
    dstc_g2 = dstc_g.reshape(-1, 128)
    srcc_g2 = srcc_g.reshape(-1, 128)
    dstc_s2 = dstc_s.reshape(-1, 128)
    saa_g2 = saa_g.reshape(-1, 128)
    daa_s2 = daa_s.reshape(-1, 128)

    h = _tc_linear(_rpad(x_l, np_l), p['W_lig'], p['b_lig'], True)
    hc = _tc_linear(_rpad(x_c, np_c), p['W_cx'], p['b_cx'], True)
    pos16 = _cpad(_rpad(pos_c, np_c), 16)
    ns8 = _cpad(_rpad(node_s, np_a), 8)
    sq8 = _cpad(_rpad(seq.astype(F32).reshape(n_a, 1), np_a), 8)
    v16 = _cpad(_rpad(node_v.reshape(n_a, 9), np_a), 16)
    w_aa = p['W_aa']
    ws8 = jnp.pad(w_aa[:6], ((0, 2), (0, 0)))
    ha = _tc_embed_aa_call(np_a, 2048)(ns8, sq8, v16, ws8, w_aa[6:26],
                                       w_aa[26:29],
                                       p['b_aa'].reshape(1, 128))

    for i in ('1', '2', '3'):
        wm = p['Wm' + i]
        w2 = jnp.concatenate([wm[:128], wm[128:256]], axis=1)
        b2 = jnp.concatenate([p['bm' + i], jnp.zeros((128,), F32)])
        ab = _tc_linear(h, w2, b2, False)
        z = _sc_gather2(ep_l, 128, 0)(ab[:, :128], ab[:, 128:],
                                      dstl_g2, srcl_g2)
        m = _tc_msg_lig_call(ep_l, 2048)(z, ea_l, wm[256:])
        s = _sc_scatter_add(ep_l, 128, np_l)(m, dstl_s2, zeros_l)
        wu = p['Wu' + i]
        h = _tc_upd_call(np_l, 2048)(h, s, s, wu[:128], wu[128:],
                                     p['bu' + i].reshape(1, 128))
    s_l, c_l = _tc_pool_call(np_l, 2048)(h, bid_l)

    for i in ('1', '2', '3'):
        we = p['We' + i]
        w2 = jnp.concatenate([we[:128], we[128:256]], axis=1)
        b2 = jnp.concatenate([p['be' + i], jnp.zeros((128,), F32)])
        ab = _tc_linear(hc, w2, b2, False)
        ta = jnp.concatenate([ab[:, :128], pos16], axis=1)
        tb = jnp.concatenate([ab[:, 128:], pos16], axis=1)
        z = _sc_gather2(ep_c, 144, 1)(ta, tb, dstc_g2, srcc_g2)
        we_e = jnp.pad(we[257:261], ((0, 4), (0, 0)))
        wx = jnp.pad(p['Wx' + i], ((0, 0), (0, 127)))
        bx = jnp.broadcast_to(p['bx' + i].reshape(1, 1), (1, 128))
        m = _tc_msg_cx_call(ep_c, 2048)(z, ea_c, we_e, we[256:257], wx, bx)
        s = _sc_scatter_add(ep_c, 144, np_c)(m, dstc_s2, zeros_c)
        wh = p['Wh' + i]
        hc, pos16 = _tc_upd_cx_call(np_c, 2048)(
            hc, pos16, s, s, wh[:128], wh[128:],
            p['bh' + i].reshape(1, 128))
    s_c, c_c = _tc_pool_call(np_c, 2048)(hc, bid_c)

    aa_t = _tc_linear(ha, p['W_amsg'][:128], p['b_amsg'], False)
    z = _sc_gather1(ep_a, 128)(aa_t, saa_g2)
    m = _tc_msg_aa_call(ep_a, 2048)(z, es_a, ev_a, p['W_amsg'][128:160],
                                    p['W_amsg'][160:161])
    s = _sc_scatter_add(ep_a, 144, np_a)(m, daa_s2, zeros_a)
    wv = p['W_aupd']
    ha = _tc_upd_aa_call(np_a, 2048)(ha, s, s, wv[:128], wv[128:],
                                     p['b_aupd'].reshape(1, 128))
    s_a, c_a = _tc_pool_call(np_a, 2048)(ha, bid_a)

    wg2 = jnp.pad(p['W_g2'], ((0, 0), (0, 127)))
    bg2 = jnp.broadcast_to(p['b_g2'].reshape(1, 1), (1, 128))
    o1, o2 = _tc_head_call()(s_l, c_l, s_a, c_a, s_c, c_c,
                             p['W_int'], p['b_int'].reshape(1, 128),
                             p['W_g1'], p['b_g1'].reshape(1, 256),
                             wg2, bg2)
    return o1[:, :1], o2[:, :1]


# speedup vs baseline: 1.1601x; 1.0769x over previous
import dataclasses
import functools

import jax
import jax.numpy as jnp
from jax import lax
from jax.experimental import pallas as pl
from jax.experimental.pallas import tpu as pltpu
from jax.experimental.pallas import tpu_sc as plsc

F32 = jnp.float32
I32 = jnp.int32
_NC, _NS, _LN = 2, 16, 16
_NW = _NC * _NS
_CHUNK = 128
_PREC = lax.Precision.DEFAULT


def _silu(x):
    return x * jax.nn.sigmoid(x)


def _dot(a, b):
    return lax.dot_general(a, b, (((1,), (0,)), ((), ())),
                           precision=_PREC, preferred_element_type=F32)


def _ceil_to(n, m):
    return -(-n // m) * m


def _rpad(x, rows):
    return jnp.pad(x, ((0, rows - x.shape[0]),) + ((0, 0),) * (x.ndim - 1))


def _cpad(x, cols):
    return jnp.pad(x, ((0, 0), (0, cols - x.shape[1])))


def _ipad(x, n, val):
    return jnp.concatenate([x.astype(I32), jnp.full((n - x.shape[0],), val, I32)])


def _blk_rows(r):
    return 2048 if r % 2048 == 0 else (1024 if r % 1024 == 0 else r)


def _sc_params():
    cp = pltpu.CompilerParams()
    if "use_tc_tiling_on_sc" in pltpu.CompilerParams.__dataclass_fields__:
        cp = dataclasses.replace(cp, use_tc_tiling_on_sc=False)
    return cp


def _sc_mesh():
    return plsc.VectorSubcoreMesh(core_axis_name="c", subcore_axis_name="s",
                                  num_cores=_NC, num_subcores=_NS)


@functools.lru_cache(None)
def _sc_gather2(e_pad, d, nsub):
    C = 128
    epw = e_pad // _NW
    nch = epw // C
    rpw = epw // 128
    ngrp = d // _LN

    def body(ta, tb, ia2, ib2, z, ia_all, ib_all,
             ra0, ra1, ra2, rb0, rb1, rb2,
             g0, g1, g2, o0, o1, o2):
        wid = lax.axis_index("s") * _NC + lax.axis_index("c")
        irow0 = wid * rpw
        pltpu.sync_copy(ia2.at[pl.ds(irow0, rpw)], ia_all)
        pltpu.sync_copy(ib2.at[pl.ds(irow0, rpw)], ib_all)
        base0 = wid * epw

        ra_v = [ra0, ra1, ra2]
        rb_v = [rb0, rb1, rb2]
        gsem = [g0, g1, g2]
        osem = [o0, o1, o2]

        def issue(b, g):
            pltpu.async_copy(ta.at[ia_all.at[g]], ra_v[b], gsem[b])
            pltpu.async_copy(tb.at[ib_all.at[g]], rb_v[b], gsem[b])

        def wait_gather(b):
            for dst in (ra_v[b], rb_v[b]):
                pltpu.make_async_copy(z.at[pl.ds(0, C)], dst, gsem[b]).wait()

        def out(b, g):
            pltpu.async_copy(ra_v[b], z.at[pl.ds(base0 + g * C, C)], osem[b])

        def wait_out(b):
            pltpu.make_async_copy(z.at[pl.ds(0, C)], ra_v[b], osem[b]).wait()

        issue(0, 0)
        issue(1, 1)

        @pl.loop(0, nch)
        def _chunk(g):
            for b in range(3):
                @pl.when(lax.rem(g, 3) == b)
                def _go():
                    nxt = (b + 2) % 3
                    wait_gather(b)

                    @pl.when(g + 2 < nch)
                    def _pf():
                        @pl.when(g >= 1)
                        def _wo():
                            wait_out(nxt)

                        issue(nxt, g + 2)

                    @pl.loop(0, C)
                    def _row(r):
                        for j in range(ngrp):
                            sl = pl.ds(j * _LN, _LN)
                            if j < ngrp - nsub:
                                ra_v[b][r, sl] = ra_v[b][r, sl] + rb_v[b][r, sl]
                            else:
                                ra_v[b][r, sl] = ra_v[b][r, sl] - rb_v[b][r, sl]

                    out(b, g)

        wait_out(0)
        wait_out(1)
        wait_out(2)

    return pl.kernel(
        body,
        out_type=jax.ShapeDtypeStruct((e_pad, d), F32),
        mesh=_sc_mesh(),
        compiler_params=_sc_params(),
        scratch_types=[
            pltpu.VMEM((rpw, 128), I32),
            pltpu.VMEM((rpw, 128), I32),
            pltpu.VMEM((C, d), F32),
            pltpu.VMEM((C, d), F32),
            pltpu.VMEM((C, d), F32),
            pltpu.VMEM((C, d), F32),
            pltpu.VMEM((C, d), F32),
            pltpu.VMEM((C, d), F32),
            pltpu.SemaphoreType.DMA,
            pltpu.SemaphoreType.DMA,
            pltpu.SemaphoreType.DMA,
            pltpu.SemaphoreType.DMA,
            pltpu.SemaphoreType.DMA,
            pltpu.SemaphoreType.DMA,
        ],
    )


@functools.lru_cache(None)
def _sc_gather1(e_pad, d):
    C = 256
    epw = e_pad // _NW
    nch = epw // C
    rpw = epw // 128

    def body(ta, ia2, z, ia_all, z0, z1, gs0, gs1, os0, os1):
        wid = lax.axis_index("s") * _NC + lax.axis_index("c")
        irow0 = wid * rpw
        pltpu.sync_copy(ia2.at[pl.ds(irow0, rpw)], ia_all)
        base0 = wid * epw
        z_v = [z0, z1]
        gsem = [gs0, gs1]
        osem = [os0, os1]

        def gather(b, g):
            for h in (0, 1):
                sl = pl.ds(h * 128, 128)
                pltpu.async_copy(ta.at[ia_all.at[2 * g + h]], z_v[b].at[sl],
                                 gsem[b])

        def wait_gather(b):
            for _ in range(2):
                pltpu.make_async_copy(z.at[pl.ds(0, 128)],
                                      z_v[b].at[pl.ds(0, 128)],
                                      gsem[b]).wait()

        def wait_out(b):
            pltpu.make_async_copy(z.at[pl.ds(0, C)], z_v[b], osem[b]).wait()

        gather(0, 0)

        @pl.loop(0, nch)
        def _chunk(g):
            b = lax.rem(g, 2)
            for bb in (0, 1):
                @pl.when(b == bb)
                def _go():
                    wait_gather(bb)

                    @pl.when(g + 1 < nch)
                    def _nx():
                        @pl.when(g >= 1)
                        def _wo():
                            wait_out(1 - bb)
                        gather(1 - bb, g + 1)

                    pltpu.async_copy(z_v[bb], z.at[pl.ds(base0 + g * C, C)],
                                     osem[bb])

        wait_out(0)
        wait_out(1)

    return pl.kernel(
        body,
        out_type=jax.ShapeDtypeStruct((e_pad, d), F32),
        mesh=_sc_mesh(),
        compiler_params=_sc_params(),
        scratch_types=[
            pltpu.VMEM((rpw, 128), I32),
            pltpu.VMEM((C, d), F32),
            pltpu.VMEM((C, d), F32),
            pltpu.SemaphoreType.DMA,
            pltpu.SemaphoreType.DMA,
            pltpu.SemaphoreType.DMA,
            pltpu.SemaphoreType.DMA,
        ],
    )


@functools.lru_cache(None)
def _sc_scatter_add(e_pad, d, n_pad):
    C = 128
    epw = e_pad // _NW
    nch = epw // C
    rpw = epw // 128
    rpt = n_pad // _NS

    def body(m, idx2, zrs, out, i0, i1, m0, m1, acc, f0, f1, s0, s1):
        c = lax.axis_index("c")
        s = lax.axis_index("s")
        wid = s * _NC + c
        row0 = s * rpt
        pltpu.sync_copy(zrs.at[pl.ds(row0, rpt)], acc.at[pl.ds(row0, rpt)])
        plsc.subcore_barrier()
        base0 = wid * epw
        irow0 = wid * rpw

        i_v = [i0, i1]
        m_v = [m0, m1]
        fsem = [f0, f1]
        ssem = [s0, s1]

        def fetch(b, g):
            pltpu.sync_copy(idx2.at[pl.ds(irow0 + g, 1)], i_v[b])
            pltpu.async_copy(m.at[pl.ds(base0 + g * C, C)], m_v[b], fsem[b])

        def wait_fetch(b):
            pltpu.make_async_copy(m.at[pl.ds(0, C)], m_v[b], fsem[b]).wait()

        def scatter(b):
            pltpu.async_copy(m_v[b], acc.at[i_v[b].at[0]], ssem[b], add=True)

        def wait_scatter(b):
            pltpu.make_async_copy(m_v[b], acc.at[i_v[b].at[0]],
                                  ssem[b]).wait()

        fetch(0, 0)

        @pl.loop(0, nch)
        def _chunk(g):
            b = lax.rem(g, 2)
            for bb in (0, 1):
                @pl.when(b == bb)
                def _go():
                    wait_fetch(bb)

                    @pl.when(g + 1 < nch)
                    def _nx():
                        @pl.when(g >= 1)
                        def _ws():
                            wait_scatter(1 - bb)
                        fetch(1 - bb, g + 1)

                    scatter(bb)

        wait_scatter(0)
        wait_scatter(1)
        plsc.subcore_barrier()
        pltpu.sync_copy(acc.at[pl.ds(row0, rpt)],
                        out.at[pl.ds(c * n_pad + row0, rpt)])

    return pl.kernel(
        body,
        out_type=jax.ShapeDtypeStruct((2 * n_pad, d), F32),
        mesh=_sc_mesh(),
        compiler_params=_sc_params(),
        scratch_types=[
            pltpu.VMEM((1, 128), I32),
            pltpu.VMEM((1, 128), I32),
            pltpu.VMEM((C, d), F32),
            pltpu.VMEM((C, d), F32),
            pltpu.VMEM_SHARED((n_pad, d), F32),
            pltpu.SemaphoreType.DMA,
            pltpu.SemaphoreType.DMA,
            pltpu.SemaphoreType.DMA,
            pltpu.SemaphoreType.DMA,
        ],
    )


@functools.lru_cache(None)
def _tc_linear_call(r, k, f, act, blk):
    def body(x_ref, w_ref, b_ref, o_ref):
        y = _dot(x_ref[...], w_ref[...]) + b_ref[...]
        o_ref[...] = _silu(y) if act else y

    return pl.pallas_call(
        body,
        grid=(r // blk,),
        in_specs=[
            pl.BlockSpec((blk, k), lambda i: (i, 0)),
            pl.BlockSpec((k, f), lambda i: (0, 0)),
            pl.BlockSpec((1, f), lambda i: (0, 0)),
        ],
        out_specs=pl.BlockSpec((blk, f), lambda i: (i, 0)),
        out_shape=jax.ShapeDtypeStruct((r, f), F32),
    )


def _tc_linear(x, w, b, act):
    r, k = x.shape
    f = w.shape[1]
    return _tc_linear_call(r, k, f, act, _blk_rows(r))(x, w, b.reshape(1, f))


@functools.lru_cache(None)
def _tc_msg_lig_call(e, blk):
    def body(z_ref, ea_ref, w_ref, o_ref):
        o_ref[...] = _silu(z_ref[...] + _dot(ea_ref[...], w_ref[...]))

    return pl.pallas_call(
        body,
        grid=(e // blk,),
        in_specs=[
            pl.BlockSpec((blk, 128), lambda i: (i, 0)),
            pl.BlockSpec((blk, 16), lambda i: (i, 0)),
            pl.BlockSpec((16, 128), lambda i: (0, 0)),
        ],
        out_specs=pl.BlockSpec((blk, 128), lambda i: (i, 0)),
        out_shape=jax.ShapeDtypeStruct((e, 128), F32),
    )


@functools.lru_cache(None)
def _tc_msg_cx_call(e, blk):
    def body(z_ref, ea_ref, we_ref, wd2_ref, wx_ref, bx_ref, o_ref):
        zz = z_ref[...]
        z128 = zz[:, :128]
        rel = zz[:, 128:131]
        d2 = jnp.sum(rel * rel, axis=1, keepdims=True)
        d2b = d2.astype(jnp.bfloat16).astype(F32)
        wd2 = wd2_ref[...].astype(jnp.bfloat16).astype(F32)
        mm = _silu(z128 + d2b * wd2 + _dot(ea_ref[...], we_ref[...]))
        coef = jnp.tanh(_dot(mm, wx_ref[...])[:, 0:1] + bx_ref[...][:, 0:1])
        relc = rel * coef
        ones = jnp.ones((blk, 1), F32)
        zer = jnp.zeros((blk, 12), F32)
        o_ref[...] = jnp.concatenate([mm, relc, ones, zer], axis=1)

    return pl.pallas_call(
        body,
        grid=(e // blk,),
        in_specs=[
            pl.BlockSpec((blk, 144), lambda i: (i, 0)),
            pl.BlockSpec((blk, 8), lambda i: (i, 0)),
            pl.BlockSpec((8, 128), lambda i: (0, 0)),
            pl.BlockSpec((1, 128), lambda i: (0, 0)),
            pl.BlockSpec((128, 128), lambda i: (0, 0)),
            pl.BlockSpec((1, 128), lambda i: (0, 0)),
        ],
        out_specs=pl.BlockSpec((blk, 144), lambda i: (i, 0)),
        out_shape=jax.ShapeDtypeStruct((e, 144), F32),
    )


@functools.lru_cache(None)
def _tc_msg_aa_call(e, blk):
    def body(z_ref, es_ref, ev_ref, we_ref, wv_ref, o_ref):
        ev = ev_ref[...][:, 0:3]
        evn = jnp.sqrt(jnp.sum(ev * ev, axis=1, keepdims=True) + 1e-8)
        evnb = evn.astype(jnp.bfloat16).astype(F32)
        wvb = wv_ref[...].astype(jnp.bfloat16).astype(F32)
        ma = _silu(z_ref[...] + _dot(es_ref[...], we_ref[...])
                   + evnb * wvb)
        ones = jnp.ones((blk, 1), F32)
        zer = jnp.zeros((blk, 15), F32)
        o_ref[...] = jnp.concatenate([ma, ones, zer], axis=1)

    return pl.pallas_call(
        body,
        grid=(e // blk,),
        in_specs=[
            pl.BlockSpec((blk, 128), lambda i: (i, 0)),
            pl.BlockSpec((blk, 32), lambda i: (i, 0)),
            pl.BlockSpec((blk, 8), lambda i: (i, 0)),
            pl.BlockSpec((32, 128), lambda i: (0, 0)),
            pl.BlockSpec((1, 128), lambda i: (0, 0)),
        ],
        out_specs=pl.BlockSpec((blk, 144), lambda i: (i, 0)),
        out_shape=jax.ShapeDtypeStruct((e, 144), F32),
    )


@functools.lru_cache(None)
def _tc_upd_call(n, blk):
    nb = n // blk

    def body(h_ref, s0_ref, s1_ref, wh_ref, wa_ref, b_ref, o_ref):
        agg = s0_ref[...] + s1_ref[...]
        o_ref[...] = _silu(_dot(h_ref[...], wh_ref[...])
                           + _dot(agg, wa_ref[...]) + b_ref[...])

    return pl.pallas_call(
        body,
        grid=(nb,),
        in_specs=[
            pl.BlockSpec((blk, 128), lambda i: (i, 0)),
            pl.BlockSpec((blk, 128), lambda i: (i, 0)),
            pl.BlockSpec((blk, 128), lambda i: (i + nb, 0)),
            pl.BlockSpec((128, 128), lambda i: (0, 0)),
            pl.BlockSpec((128, 128), lambda i: (0, 0)),
            pl.BlockSpec((1, 128), lambda i: (0, 0)),
        ],
        out_specs=pl.BlockSpec((blk, 128), lambda i: (i, 0)),
        out_shape=jax.ShapeDtypeStruct((n, 128), F32),
    )


@functools.lru_cache(None)
def _tc_upd_cx_call(n, blk):
    nb = n // blk

    def body(h_ref, p_ref, s0_ref, s1_ref, wh_ref, wa_ref, b_ref,
             oh_ref, op_ref):
        agg = s0_ref[...] + s1_ref[...]
        aggm = agg[:, :128]
        extras = agg[:, 128:144]
        cnt = extras[:, 3:4]
        inv = 1.0 / jnp.maximum(cnt, 1.0)
        lane = lax.broadcasted_iota(I32, (blk, 16), 1)
        op_ref[...] = p_ref[...] + jnp.where(lane < 3, extras * inv, 0.0)
        oh_ref[...] = _silu(_dot(h_ref[...], wh_ref[...])
                            + _dot(aggm, wa_ref[...]) + b_ref[...])

    return pl.pallas_call(
        body,
        grid=(nb,),
        in_specs=[
            pl.BlockSpec((blk, 128), lambda i: (i, 0)),
            pl.BlockSpec((blk, 16), lambda i: (i, 0)),
            pl.BlockSpec((blk, 144), lambda i: (i, 0)),
            pl.BlockSpec((blk, 144), lambda i: (i + nb, 0)),
            pl.BlockSpec((128, 128), lambda i: (0, 0)),
            pl.BlockSpec((128, 128), lambda i: (0, 0)),
            pl.BlockSpec((1, 128), lambda i: (0, 0)),
        ],
        out_specs=[
            pl.BlockSpec((blk, 128), lambda i: (i, 0)),
            pl.BlockSpec((blk, 16), lambda i: (i, 0)),
        ],
        out_shape=[
            jax.ShapeDtypeStruct((n, 128), F32),
            jax.ShapeDtypeStruct((n, 16), F32),
        ],
    )


@functools.lru_cache(None)
def _tc_upd_aa_call(n, blk):
    nb = n // blk

    def body(h_ref, s0_ref, s1_ref, wh_ref, wa_ref, b_ref, o_ref):
        agg = s0_ref[...] + s1_ref[...]
        cnt = agg[:, 128:129]
        aggmean = agg[:, :128] / jnp.maximum(cnt, 1.0)
        o_ref[...] = _silu(_dot(h_ref[...], wh_ref[...])
                           + _dot(aggmean, wa_ref[...]) + b_ref[...])

    return pl.pallas_call(
        body,
        grid=(nb,),
        in_specs=[
            pl.BlockSpec((blk, 128), lambda i: (i, 0)),
            pl.BlockSpec((blk, 144), lambda i: (i, 0)),
            pl.BlockSpec((blk, 144), lambda i: (i + nb, 0)),
            pl.BlockSpec((128, 128), lambda i: (0, 0)),
            pl.BlockSpec((128, 128), lambda i: (0, 0)),
            pl.BlockSpec((1, 128), lambda i: (0, 0)),
        ],
        out_specs=pl.BlockSpec((blk, 128), lambda i: (i, 0)),
        out_shape=jax.ShapeDtypeStruct((n, 128), F32),
    )


@functools.lru_cache(None)
def _tc_embed_aa_call(n, blk):
    def body(s8_ref, sq_ref, v16_ref, ws_ref, woh_ref, wv_ref, b_ref, o_ref):
        sq = sq_ref[...][:, 0:1]
        i20 = lax.broadcasted_iota(I32, (1, 20), 1).astype(F32)
        oh = (sq == i20).astype(F32)
        v = v16_ref[...]

        def nrm(k):
            sl = v[:, 3 * k:3 * k + 3]
            return jnp.sqrt(jnp.sum(sl * sl, axis=1, keepdims=True) + 1e-8)

        vn = jnp.concatenate([nrm(0), nrm(1), nrm(2)], axis=1)
        o_ref[...] = _silu(_dot(s8_ref[...], ws_ref[...])
                           + _dot(oh, woh_ref[...])
                           + _dot(vn, wv_ref[...]) + b_ref[...])

    return pl.pallas_call(
        body,
        grid=(n // blk,),
        in_specs=[
            pl.BlockSpec((blk, 8), lambda i: (i, 0)),
            pl.BlockSpec((blk, 8), lambda i: (i, 0)),
            pl.BlockSpec((blk, 16), lambda i: (i, 0)),
            pl.BlockSpec((8, 128), lambda i: (0, 0)),
            pl.BlockSpec((20, 128), lambda i: (0, 0)),
            pl.BlockSpec((3, 128), lambda i: (0, 0)),
            pl.BlockSpec((1, 128), lambda i: (0, 0)),
        ],
        out_specs=pl.BlockSpec((blk, 128), lambda i: (i, 0)),
        out_shape=jax.ShapeDtypeStruct((n, 128), F32),
    )


@functools.lru_cache(None)
def _tc_eea_call(e, blk):
    def body(es_ref, ev_ref, we_ref, wv_ref, o_ref):
        ev = ev_ref[...][:, 0:3]
        evn = jnp.sqrt(jnp.sum(ev * ev, axis=1, keepdims=True) + 1e-8)
        evnb = evn.astype(jnp.bfloat16).astype(F32)
        wvb = wv_ref[...].astype(jnp.bfloat16).astype(F32)
        o_ref[...] = _dot(es_ref[...], we_ref[...]) + evnb * wvb

    return pl.pallas_call(
        body,
        grid=(e // blk,),
        in_specs=[
            pl.BlockSpec((blk, 32), lambda i: (i, 0)),
            pl.BlockSpec((blk, 8), lambda i: (i, 0)),
            pl.BlockSpec((32, 128), lambda i: (0, 0)),
            pl.BlockSpec((1, 128), lambda i: (0, 0)),
        ],
        out_specs=pl.BlockSpec((blk, 128), lambda i: (i, 0)),
        out_shape=jax.ShapeDtypeStruct((e, 128), F32),
    )


@functools.lru_cache(None)
def _tc_pool_call(n, blk):
    def body(x_ref, bid_ref, s_ref, c_ref):
        @pl.when(pl.program_id(0) == 0)
        def _init():
            s_ref[...] = jnp.zeros_like(s_ref)
            c_ref[...] = jnp.zeros_like(c_ref)

        bid = bid_ref[...]
        i16 = lax.broadcasted_iota(I32, (16, blk), 0)
        one_t = (i16 == bid).astype(F32)
        s_ref[...] += lax.dot_general(one_t, x_ref[...],
                                      (((1,), (0,)), ((), ())),
                                      precision=lax.Precision.HIGHEST,
                                      preferred_element_type=F32)
        csum = jnp.sum(one_t, axis=1, keepdims=True)
        c_ref[...] += jnp.broadcast_to(csum, (16, 128))

    return pl.pallas_call(
        body,
        grid=(n // blk,),
        in_specs=[
            pl.BlockSpec((blk, 128), lambda i: (i, 0)),
            pl.BlockSpec((1, blk), lambda i: (0, i)),
        ],
        out_specs=[
            pl.BlockSpec((16, 128), lambda i: (0, 0)),
            pl.BlockSpec((16, 128), lambda i: (0, 0)),
        ],
        out_shape=[
            jax.ShapeDtypeStruct((16, 128), F32),
            jax.ShapeDtypeStruct((16, 128), F32),
        ],
    )


@functools.lru_cache(None)
def _tc_head_call():
    def body(sl_ref, cl_ref, sa_ref, ca_ref, sc_ref, cc_ref,
             wi_ref, bi_ref, wg1_ref, bg1_ref, wg2_ref, bg2_ref,
             o1_ref, o2_ref):
        p_l = sl_ref[...] / jnp.maximum(cl_ref[...], 1.0)
        p_a = sa_ref[...] / jnp.maximum(ca_ref[...], 1.0)
        x_c = sc_ref[...] / jnp.maximum(cc_ref[...], 1.0)
        inter = _silu(_dot(jnp.concatenate([p_l, p_a], axis=1), wi_ref[...])
                      + bi_ref[...])

        def g(v):
            h1 = jnp.maximum(_dot(v, wg1_ref[...]) + bg1_ref[...], 0.0)
            return _dot(h1, wg2_ref[...]) + bg2_ref[...]

        o1_ref[...] = g(inter)
        o2_ref[...] = g(x_c)

    return pl.pallas_call(
        body,
        out_shape=[
            jax.ShapeDtypeStruct((16, 128), F32),
            jax.ShapeDtypeStruct((16, 128), F32),
        ],
    )


def kernel(x_l, edge_attr_l, x_c, pos_c, edge_attr_c, node_s, node_v,
           edge_s, edge_v, params, edge_index_l, batch_l, edge_index_c,
           batch_c, seq, edge_index_aa, batch_aa):
    p = params
    n_l, n_c, n_a = x_l.shape[0], x_c.shape[0], node_s.shape[0]
    e_l, e_c, e_a = (edge_index_l.shape[1], edge_index_c.shape[1],
                     edge_index_aa.shape[1])
    np_l, np_c, np_a = (_ceil_to(n_l, 2048), _ceil_to(n_c, 2048),
                        _ceil_to(n_a, 2048))
    ep_l, ep_c, ep_a = (_ceil_to(e_l, 4096), _ceil_to(e_c, 4096),
                        _ceil_to(e_a, 4096))

    srcl_g = _ipad(edge_index_l[0], ep_l, 0)
    dstl_g = _ipad(edge_index_l[1], ep_l, 0)
    dstl_s = _ipad(edge_index_l[1], ep_l, n_l)
    srcc_g = _ipad(edge_index_c[0], ep_c, 0)
    dstc_g = _ipad(edge_index_c[1], ep_c, 0)
    dstc_s = _ipad(edge_index_c[1], ep_c, n_c)
    saa_g = _ipad(edge_index_aa[0], ep_a, 0)
    daa_s = _ipad(edge_index_aa[1], ep_a, n_a)

    ea_l = _rpad(edge_attr_l, ep_l)
    ea_c = _cpad(_rpad(edge_attr_c, ep_c), 8)
    es_a = _rpad(edge_s, ep_a)
    ev_a = _cpad(_rpad(edge_v.reshape(e_a, 3), ep_a), 8)

    bid_l = _ipad(batch_l, np_l, 63).reshape(1, np_l)
    bid_c = _ipad(batch_c, np_c, 63).reshape(1, np_c)
    bid_a = _ipad(batch_aa, np_a, 63).reshape(1, np_a)

    zeros_l = jnp.zeros((np_l, 128), F32)
    zeros_c = jnp.zeros((np_c, 144), F32)
    zeros_a = jnp.zeros((np_a, 144), F32)

    def dep(idx_arr, t):
        if t is None:
            return idx_arr
        out, _ = lax.optimization_barrier((idx_arr, t))
        return out

    tok = None

    dstl_g2 = dstl_g.reshape(-1, 128)
    srcl_g2 = srcl_g.reshape(-1, 128)
    dstl_s2 = dstl_s.reshape(-1, 128)
    dstc_g2 = dstc_g.reshape(-1, 128)
    srcc_g2 = srcc_g.reshape(-1, 128)
    dstc_s2 = dstc_s.reshape(-1, 128)
    saa_g2 = saa_g.reshape(-1, 128)
    daa_s2 = daa_s.reshape(-1, 128)

    h = _tc_linear(_rpad(x_l, np_l), p['W_lig'], p['b_lig'], True)
    hc = _tc_linear(_rpad(x_c, np_c), p['W_cx'], p['b_cx'], True)
    pos16 = _cpad(_rpad(pos_c, np_c), 16)
    ns8 = _cpad(_rpad(node_s, np_a), 8)
    sq8 = _cpad(_rpad(seq.astype(F32).reshape(n_a, 1), np_a), 8)
    v16 = _cpad(_rpad(node_v.reshape(n_a, 9), np_a), 16)
    w_aa = p['W_aa']
    ws8 = jnp.pad(w_aa[:6], ((0, 2), (0, 0)))
    ha = _tc_embed_aa_call(np_a, 2048)(ns8, sq8, v16, ws8, w_aa[6:26],
                                       w_aa[26:29],
                                       p['b_aa'].reshape(1, 128))

    for i in ('1', '2', '3'):
        wm = p['Wm' + i]
        w2 = jnp.concatenate([wm[:128], wm[128:256]], axis=1)
        b2 = jnp.concatenate([p['bm' + i], jnp.zeros((128,), F32)])
        ab = _tc_linear(h, w2, b2, False)
        z = _sc_gather2(ep_l, 128, 0)(ab[:, :128], ab[:, 128:],
                                      dstl_g2, srcl_g2)
        m = _tc_msg_lig_call(ep_l, 2048)(z, ea_l, wm[256:])
        s = _sc_scatter_add(ep_l, 128, np_l)(m, dstl_s2, zeros_l)
        wu = p['Wu' + i]
        h = _tc_upd_call(np_l, 2048)(h, s, s, wu[:128], wu[128:],
                                     p['bu' + i].reshape(1, 128))
    s_l, c_l = _tc_pool_call(np_l, 2048)(h, bid_l)

    for i in ('1', '2', '3'):
        we = p['We' + i]
        w2 = jnp.concatenate([we[:128], we[128:256]], axis=1)
        b2 = jnp.concatenate([p['be' + i], jnp.zeros((128,), F32)])
        ab = _tc_linear(hc, w2, b2, False)
        ta = jnp.concatenate([ab[:, :128], pos16], axis=1)
        tb = jnp.concatenate([ab[:, 128:], pos16], axis=1)
        z = _sc_gather2(ep_c, 144, 1)(ta, tb, dstc_g2, srcc_g2)
        we_e = jnp.pad(we[257:261], ((0, 4), (0, 0)))
        wx = jnp.pad(p['Wx' + i], ((0, 0), (0, 127)))
        bx = jnp.broadcast_to(p['bx' + i].reshape(1, 1), (1, 128))
        m = _tc_msg_cx_call(ep_c, 2048)(z, ea_c, we_e, we[256:257], wx, bx)
        s = _sc_scatter_add(ep_c, 144, np_c)(m, dstc_s2, zeros_c)
        wh = p['Wh' + i]
        hc, pos16 = _tc_upd_cx_call(np_c, 2048)(
            hc, pos16, s, s, wh[:128], wh[128:],
            p['bh' + i].reshape(1, 128))
    s_c, c_c = _tc_pool_call(np_c, 2048)(hc, bid_c)

    aa_t = _tc_linear(ha, p['W_amsg'][:128], p['b_amsg'], False)
    z = _sc_gather1(ep_a, 128)(aa_t, saa_g2)
    m = _tc_msg_aa_call(ep_a, 2048)(z, es_a, ev_a, p['W_amsg'][128:160],
                                    p['W_amsg'][160:161])
    s = _sc_scatter_add(ep_a, 144, np_a)(m, daa_s2, zeros_a)
    wv = p['W_aupd']
    ha = _tc_upd_aa_call(np_a, 2048)(ha, s, s, wv[:128], wv[128:],
                                     p['b_aupd'].reshape(1, 128))
    s_a, c_a = _tc_pool_call(np_a, 2048)(ha, bid_a)

    wg2 = jnp.pad(p['W_g2'], ((0, 0), (0, 127)))
    bg2 = jnp.broadcast_to(p['b_g2'].reshape(1, 1), (1, 128))
    o1, o2 = _tc_head_call()(s_l, c_l, s_a, c_a, s_c, c_c,
                             p['W_int'], p['b_int'].reshape(1, 128),
                             p['W_g1'], p['b_g1'].reshape(1, 256),
                             wg2, bg2)
    return o1[:, :1], o2[:, :1]
